# baseline (speedup 1.0000x reference)
import functools

import jax
import jax.numpy as jnp
from jax import lax
from jax.experimental import pallas as pl
from jax.experimental.pallas import tpu as pltpu
from jax.experimental.pallas import tpu_sc as plsc

T = 2048
H = 1024
E = 8
K = 2
INTER = 512
SI = 1024
BM = 256
NBLK = T * K // BM + E
NR = NBLK * BM
MW = NBLK + 1
NC = 2
NS = 16
NW = NC * NS
TPW = T // NW
TBLK = 512


def _router_body(x_ref, wg_ref, b_ref, pos0_ref, pos1_ref, w0_ref, w1_ref,
                 meta_ref):
    x = x_ref[...]
    scores = jax.nn.sigmoid(
        jnp.dot(x, wg_ref[...], preferred_element_type=jnp.float32))
    biased = scores + b_ref[...]
    iota = lax.broadcasted_iota(jnp.int32, (T, E), 1)
    m0 = jnp.max(biased, axis=1, keepdims=True)
    sel0 = jnp.min(jnp.where(biased >= m0, iota, E), axis=1, keepdims=True)
    neg = jnp.where(iota == sel0, -jnp.inf, biased)
    m1 = jnp.max(neg, axis=1, keepdims=True)
    sel1 = jnp.min(jnp.where(neg >= m1, iota, E), axis=1, keepdims=True)
    s0 = jnp.sum(jnp.where(iota == sel0, scores, 0.0), axis=1, keepdims=True)
    s1 = jnp.sum(jnp.where(iota == sel1, scores, 0.0), axis=1, keepdims=True)
    denom = s0 + s1 + 1e-20
    w0_ref[...] = jnp.broadcast_to(s0 / denom, (T, 16))
    w1_ref[...] = jnp.broadcast_to(s1 / denom, (T, 16))

    memb = jnp.logical_or(iota == sel0, iota == sel1).astype(jnp.bfloat16)
    rr = lax.broadcasted_iota(jnp.int32, (T, T), 0)
    cc = lax.broadcasted_iota(jnp.int32, (T, T), 1)
    tri = (rr >= cc).astype(jnp.bfloat16)
    csum = jnp.dot(tri, memb, preferred_element_type=jnp.float32)
    counts = csum[T - 1:T, :]
    cnt_pad = jnp.floor((counts + (BM - 1)) / BM) * BM
    er = lax.broadcasted_iota(jnp.int32, (E, E), 0)
    ec = lax.broadcasted_iota(jnp.int32, (E, E), 1)
    upper = (er < ec).astype(jnp.float32)
    offs = jnp.dot(cnt_pad, upper, preferred_element_type=jnp.float32)

    posf0 = jnp.sum(jnp.where(iota == sel0, offs + csum - 1.0, 0.0),
                    axis=1, keepdims=True)
    posf1 = jnp.sum(jnp.where(iota == sel1, offs + csum - 1.0, 0.0),
                    axis=1, keepdims=True)
    pos0_ref[...] = posf0.astype(jnp.int32)
    pos1_ref[...] = posf1.astype(jnp.int32)

    evec = lax.broadcasted_iota(jnp.int32, (1, E), 1)
    off_s = [jnp.sum(jnp.where(evec == e, offs, 0.0)) for e in range(E)]
    cnt_s = [jnp.sum(jnp.where(evec == e, cnt_pad, 0.0)) for e in range(E)]
    nb_total = ((off_s[E - 1] + cnt_s[E - 1]) / BM).astype(jnp.int32)
    bio = lax.broadcasted_iota(jnp.int32, (1, MW), 1)
    bvals = (bio - 1) * BM
    be = sum(((bvals.astype(jnp.float32) >= off_s[e]).astype(jnp.int32))
             for e in range(E)) - 1
    meta_ref[...] = jnp.where(bio == 0, nb_total, be)


def _run_router(x, W_gate, expert_bias):
    return pl.pallas_call(
        _router_body,
        out_shape=(
            jax.ShapeDtypeStruct((T, 1), jnp.int32),
            jax.ShapeDtypeStruct((T, 1), jnp.int32),
            jax.ShapeDtypeStruct((T, 16), jnp.float32),
            jax.ShapeDtypeStruct((T, 16), jnp.float32),
            jax.ShapeDtypeStruct((1, MW), jnp.int32),
        ),
        in_specs=[
            pl.BlockSpec((T, H), lambda: (0, 0)),
            pl.BlockSpec((H, E), lambda: (0, 0)),
            pl.BlockSpec((1, E), lambda: (0, 0)),
        ],
        out_specs=(
            pl.BlockSpec((T, 1), lambda: (0, 0)),
            pl.BlockSpec((T, 1), lambda: (0, 0)),
            pl.BlockSpec((T, 16), lambda: (0, 0)),
            pl.BlockSpec((T, 16), lambda: (0, 0)),
            pl.BlockSpec((1, MW), lambda: (0, 0)),
        ),
    )(x, W_gate, expert_bias.reshape(1, E))


H2 = H // 2


def _dispatch_body(xb_hbm, p0_hbm, p1_hbm, xs_hbm, xrows, p0v, p1v, sem0, sem1):
    wid = lax.axis_index("s") * NC + lax.axis_index("c")
    base = wid * TPW
    pltpu.sync_copy(xb_hbm.at[pl.ds(base, TPW)], xrows)
    pltpu.sync_copy(p0_hbm.at[pl.ds(base, TPW)], p0v)
    pltpu.sync_copy(p1_hbm.at[pl.ds(base, TPW)], p1v)
    a = pltpu.async_copy(xrows, xs_hbm.at[p0v], sem0)
    b = pltpu.async_copy(xrows, xs_hbm.at[p1v], sem1)
    a.wait()
    b.wait()


def _run_dispatch(xb_i32, pos0, pos1):
    mesh = plsc.VectorSubcoreMesh(core_axis_name="c", subcore_axis_name="s")
    f = functools.partial(
        pl.kernel,
        out_type=jax.ShapeDtypeStruct((NR, H2), jnp.int32),
        mesh=mesh,
        scratch_types=[
            pltpu.VMEM((TPW, H2), jnp.int32),
            pltpu.VMEM((TPW,), jnp.int32),
            pltpu.VMEM((TPW,), jnp.int32),
            pltpu.SemaphoreType.DMA,
            pltpu.SemaphoreType.DMA,
        ],
    )(_dispatch_body)
    return f(xb_i32, pos0, pos1)


def _grouped_body(m_ref, xs_ref, wg_ref, wu_ref, wd_ref, ys_ref):
    b = pl.program_id(0)

    @pl.when(b < m_ref[0])
    def _():
        xb = xs_ref[...]
        hg = jnp.dot(xb, wg_ref[0], preferred_element_type=jnp.float32)
        hu = jnp.dot(xb, wu_ref[0], preferred_element_type=jnp.float32)
        mid = (jax.nn.silu(hg) * hu).astype(jnp.bfloat16)
        ys_ref[...] = jnp.dot(
            mid, wd_ref[0], preferred_element_type=jnp.float32
        ).astype(jnp.bfloat16)


def _run_grouped(meta1d, xs, Wgb, Wub, Wdb):
    grid_spec = pltpu.PrefetchScalarGridSpec(
        num_scalar_prefetch=1,
        grid=(NBLK,),
        in_specs=[
            pl.BlockSpec((BM, H), lambda b, m: (b, 0)),
            pl.BlockSpec((1, H, INTER), lambda b, m: (m[b + 1], 0, 0)),
            pl.BlockSpec((1, H, INTER), lambda b, m: (m[b + 1], 0, 0)),
            pl.BlockSpec((1, INTER, H), lambda b, m: (m[b + 1], 0, 0)),
        ],
        out_specs=pl.BlockSpec((BM, H), lambda b, m: (b, 0)),
    )
    return pl.pallas_call(
        _grouped_body,
        grid_spec=grid_spec,
        out_shape=jax.ShapeDtypeStruct((NR, H), jnp.bfloat16),
    )(meta1d, xs, Wgb, Wub, Wdb)


def _gather_body(ys_hbm, p0_hbm, p1_hbm, y0g_hbm, y1g_hbm,
                 y0b, y1b, p0v, p1v, sem0, sem1):
    wid = lax.axis_index("s") * NC + lax.axis_index("c")
    base = wid * TPW
    pltpu.sync_copy(p0_hbm.at[pl.ds(base, TPW)], p0v)
    pltpu.sync_copy(p1_hbm.at[pl.ds(base, TPW)], p1v)
    a = pltpu.async_copy(ys_hbm.at[p0v], y0b, sem0)
    b = pltpu.async_copy(ys_hbm.at[p1v], y1b, sem1)
    a.wait()
    b.wait()
    pltpu.sync_copy(y0b, y0g_hbm.at[pl.ds(base, TPW)])
    pltpu.sync_copy(y1b, y1g_hbm.at[pl.ds(base, TPW)])


def _run_gather(ys_i32, pos0, pos1):
    mesh = plsc.VectorSubcoreMesh(core_axis_name="c", subcore_axis_name="s")
    f = functools.partial(
        pl.kernel,
        out_type=(
            jax.ShapeDtypeStruct((T, H2), jnp.int32),
            jax.ShapeDtypeStruct((T, H2), jnp.int32),
        ),
        mesh=mesh,
        scratch_types=[
            pltpu.VMEM((TPW, H2), jnp.int32),
            pltpu.VMEM((TPW, H2), jnp.int32),
            pltpu.VMEM((TPW,), jnp.int32),
            pltpu.VMEM((TPW,), jnp.int32),
            pltpu.SemaphoreType.DMA,
            pltpu.SemaphoreType.DMA,
        ],
    )(_gather_body)
    return f(ys_i32, pos0, pos1)


def _shared_body(xb_ref, wgs_ref, wus_ref, wds_ref, y0_ref, y1_ref,
                 w0_ref, w1_ref, out_ref):
    xb = xb_ref[...]
    hg = jnp.dot(xb, wgs_ref[...], preferred_element_type=jnp.float32)
    hu = jnp.dot(xb, wus_ref[...], preferred_element_type=jnp.float32)
    mid = (jax.nn.silu(hg) * hu).astype(jnp.bfloat16)
    sh = jnp.dot(mid, wds_ref[...], preferred_element_type=jnp.float32)
    w0 = w0_ref[...][:, 0:1]
    w1 = w1_ref[...][:, 0:1]
    out_ref[...] = (sh + w0 * y0_ref[...].astype(jnp.float32)
                    + w1 * y1_ref[...].astype(jnp.float32))


def _run_shared_combine(xb, Wg_s, Wu_s, Wd_s, y0g, y1g, w0, w1):
    return pl.pallas_call(
        _shared_body,
        grid=(T // TBLK,),
        out_shape=jax.ShapeDtypeStruct((T, H), jnp.float32),
        in_specs=[
            pl.BlockSpec((TBLK, H), lambda t: (t, 0)),
            pl.BlockSpec((H, SI), lambda t: (0, 0)),
            pl.BlockSpec((H, SI), lambda t: (0, 0)),
            pl.BlockSpec((SI, H), lambda t: (0, 0)),
            pl.BlockSpec((TBLK, H), lambda t: (t, 0)),
            pl.BlockSpec((TBLK, H), lambda t: (t, 0)),
            pl.BlockSpec((TBLK, 16), lambda t: (t, 0)),
            pl.BlockSpec((TBLK, 16), lambda t: (t, 0)),
        ],
        out_specs=pl.BlockSpec((TBLK, H), lambda t: (t, 0)),
    )(xb, Wg_s, Wu_s, Wd_s, y0g, y1g, w0, w1)


def kernel(hidden_states, W_gate, Wg_s, Wu_s, Wd_s, Wg, Wu, Wd, expert_bias):
    b, s, h = hidden_states.shape
    x = hidden_states.reshape(T, H)
    xb = x.astype(jnp.bfloat16)

    pos0, pos1, w0, w1, meta = _run_router(x, W_gate, expert_bias)
    pos0 = pos0.reshape(T)
    pos1 = pos1.reshape(T)
    meta1d = meta.reshape(MW)

    xb_i32 = lax.bitcast_convert_type(
        xb.reshape(T, H2, 2), jnp.int32)
    xs_i32 = _run_dispatch(xb_i32, pos0, pos1)
    xs = lax.bitcast_convert_type(xs_i32, jnp.bfloat16).reshape(NR, H)
    ys = _run_grouped(meta1d, xs, Wg.astype(jnp.bfloat16),
                      Wu.astype(jnp.bfloat16), Wd.astype(jnp.bfloat16))
    ys_i32 = lax.bitcast_convert_type(ys.reshape(NR, H2, 2), jnp.int32)
    y0g_i32, y1g_i32 = _run_gather(ys_i32, pos0, pos1)
    y0g = lax.bitcast_convert_type(y0g_i32, jnp.bfloat16).reshape(T, H)
    y1g = lax.bitcast_convert_type(y1g_i32, jnp.bfloat16).reshape(T, H)
    out = _run_shared_combine(xb, Wg_s.astype(jnp.bfloat16),
                              Wu_s.astype(jnp.bfloat16),
                              Wd_s.astype(jnp.bfloat16), y0g, y1g, w0, w1)
    return out.reshape(b, s, h)


# speedup vs baseline: 4.1037x; 4.1037x over previous
import functools

import jax
import jax.numpy as jnp
from jax import lax
from jax.experimental import pallas as pl
from jax.experimental.pallas import tpu as pltpu
from jax.experimental.pallas import tpu_sc as plsc

T = 2048
H = 1024
H2 = H // 2
E = 8
K = 2
INTER = 512
SI = 1024
BM = 256
NBLK = T * K // BM + E
NR = NBLK * BM
MW = NBLK + 1
NC = 2
NS = 16
NW = NC * NS
TPW = T // NW
TBLK = 512

def _pack(lo_f32, hi_f32):
    mask = jnp.full(lo_f32.shape, 0xFFFF0000, jnp.uint32)
    lo = pltpu.bitcast(lo_f32.astype(jnp.bfloat16).astype(jnp.float32),
                       jnp.uint32) >> 16
    hi = pltpu.bitcast(hi_f32.astype(jnp.bfloat16).astype(jnp.float32),
                       jnp.uint32) & mask
    return pltpu.bitcast(lo | hi, jnp.int32)


def _unpack(word_i32):
    mask = jnp.full(word_i32.shape, 0xFFFF0000, jnp.uint32)
    w = pltpu.bitcast(word_i32, jnp.uint32)
    lo = pltpu.bitcast(w << 16, jnp.float32)
    hi = pltpu.bitcast(w & mask, jnp.float32)
    return lo, hi


def _router_body(x_ref, wg_ref, b_ref, pos0_ref, pos1_ref, w0_ref, w1_ref,
                 meta_ref, xpk_ref):
    x = x_ref[...]
    xpk_ref[...] = _pack(x[:, :H2], x[:, H2:])
    scores = jax.nn.sigmoid(
        jnp.dot(x, wg_ref[...], preferred_element_type=jnp.float32))
    biased = scores + b_ref[...]
    iota = lax.broadcasted_iota(jnp.int32, (T, E), 1)
    m0 = jnp.max(biased, axis=1, keepdims=True)
    sel0 = jnp.min(jnp.where(biased >= m0, iota, E), axis=1, keepdims=True)
    neg = jnp.where(iota == sel0, -jnp.inf, biased)
    m1 = jnp.max(neg, axis=1, keepdims=True)
    sel1 = jnp.min(jnp.where(neg >= m1, iota, E), axis=1, keepdims=True)
    s0 = jnp.sum(jnp.where(iota == sel0, scores, 0.0), axis=1, keepdims=True)
    s1 = jnp.sum(jnp.where(iota == sel1, scores, 0.0), axis=1, keepdims=True)
    denom = s0 + s1 + 1e-20
    w0_ref[...] = jnp.broadcast_to(s0 / denom, (T, 16))
    w1_ref[...] = jnp.broadcast_to(s1 / denom, (T, 16))

    memb = jnp.logical_or(iota == sel0, iota == sel1).astype(jnp.bfloat16)
    rr = lax.broadcasted_iota(jnp.int32, (T, T), 0)
    cc = lax.broadcasted_iota(jnp.int32, (T, T), 1)
    tri = (rr >= cc).astype(jnp.bfloat16)
    csum = jnp.dot(tri, memb, preferred_element_type=jnp.float32)
    counts = csum[T - 1:T, :]
    cnt_pad = jnp.floor((counts + (BM - 1)) / BM) * BM
    er = lax.broadcasted_iota(jnp.int32, (E, E), 0)
    ec = lax.broadcasted_iota(jnp.int32, (E, E), 1)
    upper = (er < ec).astype(jnp.float32)
    offs = jnp.dot(cnt_pad, upper, preferred_element_type=jnp.float32)

    posf0 = jnp.sum(jnp.where(iota == sel0, offs + csum - 1.0, 0.0),
                    axis=1, keepdims=True)
    posf1 = jnp.sum(jnp.where(iota == sel1, offs + csum - 1.0, 0.0),
                    axis=1, keepdims=True)
    pos0_ref[...] = posf0.astype(jnp.int32)
    pos1_ref[...] = posf1.astype(jnp.int32)

    evec = lax.broadcasted_iota(jnp.int32, (1, E), 1)
    off_s = [jnp.sum(jnp.where(evec == e, offs, 0.0)) for e in range(E)]
    cnt_s = [jnp.sum(jnp.where(evec == e, cnt_pad, 0.0)) for e in range(E)]
    nb_total = ((off_s[E - 1] + cnt_s[E - 1]) / BM).astype(jnp.int32)
    bio = lax.broadcasted_iota(jnp.int32, (1, MW), 1)
    bvals = (bio - 1) * BM
    be = sum(((bvals.astype(jnp.float32) >= off_s[e]).astype(jnp.int32))
             for e in range(E)) - 1
    meta_ref[...] = jnp.where(bio == 0, nb_total, be)


def _run_router(x, W_gate, expert_bias):
    return pl.pallas_call(
        _router_body,
        out_shape=(
            jax.ShapeDtypeStruct((T, 1), jnp.int32),
            jax.ShapeDtypeStruct((T, 1), jnp.int32),
            jax.ShapeDtypeStruct((T, 16), jnp.float32),
            jax.ShapeDtypeStruct((T, 16), jnp.float32),
            jax.ShapeDtypeStruct((1, MW), jnp.int32),
            jax.ShapeDtypeStruct((T, H2), jnp.int32),
        ),
        in_specs=[
            pl.BlockSpec((T, H), lambda: (0, 0)),
            pl.BlockSpec((H, E), lambda: (0, 0)),
            pl.BlockSpec((1, E), lambda: (0, 0)),
        ],
        out_specs=(
            pl.BlockSpec((T, 1), lambda: (0, 0)),
            pl.BlockSpec((T, 1), lambda: (0, 0)),
            pl.BlockSpec((T, 16), lambda: (0, 0)),
            pl.BlockSpec((T, 16), lambda: (0, 0)),
            pl.BlockSpec((1, MW), lambda: (0, 0)),
            pl.BlockSpec((T, H2), lambda: (0, 0)),
        ),
    )(x, W_gate, expert_bias.reshape(1, E))


def _dispatch_body(xpk_hbm, p0_hbm, p1_hbm, xs_hbm, xrows, p0v, p1v,
                   sem0, sem1):
    wid = lax.axis_index("s") * NC + lax.axis_index("c")
    base = wid * TPW
    pltpu.sync_copy(xpk_hbm.at[pl.ds(base, TPW)], xrows)
    pltpu.sync_copy(p0_hbm.at[pl.ds(base, TPW)], p0v)
    pltpu.sync_copy(p1_hbm.at[pl.ds(base, TPW)], p1v)
    a = pltpu.async_copy(xrows, xs_hbm.at[p0v], sem0)
    b = pltpu.async_copy(xrows, xs_hbm.at[p1v], sem1)
    a.wait()
    b.wait()


def _run_dispatch(xpk, pos0, pos1):
    mesh = plsc.VectorSubcoreMesh(core_axis_name="c", subcore_axis_name="s")
    f = functools.partial(
        pl.kernel,
        out_type=jax.ShapeDtypeStruct((NR, H2), jnp.int32),
        mesh=mesh,
        scratch_types=[
            pltpu.VMEM((TPW, H2), jnp.int32),
            pltpu.VMEM((TPW,), jnp.int32),
            pltpu.VMEM((TPW,), jnp.int32),
            pltpu.SemaphoreType.DMA,
            pltpu.SemaphoreType.DMA,
        ],
    )(_dispatch_body)
    return f(xpk, pos0, pos1)


def _grouped_body(m_ref, xs_ref, wg_ref, wu_ref, wd_ref, ys_ref):
    b = pl.program_id(0)

    @pl.when(b < m_ref[0])
    def _():
        xa, xb_ = _unpack(xs_ref[...])
        xa = xa.astype(jnp.bfloat16)
        xb_ = xb_.astype(jnp.bfloat16)
        hg = (jnp.dot(xa, wg_ref[0, :H2, :], preferred_element_type=jnp.float32)
              + jnp.dot(xb_, wg_ref[0, H2:, :],
                        preferred_element_type=jnp.float32))
        hu = (jnp.dot(xa, wu_ref[0, :H2, :], preferred_element_type=jnp.float32)
              + jnp.dot(xb_, wu_ref[0, H2:, :],
                        preferred_element_type=jnp.float32))
        mid = (jax.nn.silu(hg) * hu).astype(jnp.bfloat16)
        ys = jnp.dot(mid, wd_ref[0], preferred_element_type=jnp.float32)
        ys_ref[...] = _pack(ys[:, :H2], ys[:, H2:])


def _run_grouped(meta1d, xs, Wgb, Wub, Wdb):
    grid_spec = pltpu.PrefetchScalarGridSpec(
        num_scalar_prefetch=1,
        grid=(NBLK,),
        in_specs=[
            pl.BlockSpec((BM, H2), lambda b, m: (b, 0)),
            pl.BlockSpec((1, H, INTER), lambda b, m: (m[b + 1], 0, 0)),
            pl.BlockSpec((1, H, INTER), lambda b, m: (m[b + 1], 0, 0)),
            pl.BlockSpec((1, INTER, H), lambda b, m: (m[b + 1], 0, 0)),
        ],
        out_specs=pl.BlockSpec((BM, H2), lambda b, m: (b, 0)),
    )
    return pl.pallas_call(
        _grouped_body,
        grid_spec=grid_spec,
        out_shape=jax.ShapeDtypeStruct((NR, H2), jnp.int32),
    )(meta1d, xs, Wgb, Wub, Wdb)


def _gather_body(ys_hbm, p0_hbm, p1_hbm, y0g_hbm, y1g_hbm,
                 y0b, y1b, p0v, p1v, sem0, sem1):
    wid = lax.axis_index("s") * NC + lax.axis_index("c")
    base = wid * TPW
    pltpu.sync_copy(p0_hbm.at[pl.ds(base, TPW)], p0v)
    pltpu.sync_copy(p1_hbm.at[pl.ds(base, TPW)], p1v)
    a = pltpu.async_copy(ys_hbm.at[p0v], y0b, sem0)
    b = pltpu.async_copy(ys_hbm.at[p1v], y1b, sem1)
    a.wait()
    b.wait()
    pltpu.sync_copy(y0b, y0g_hbm.at[pl.ds(base, TPW)])
    pltpu.sync_copy(y1b, y1g_hbm.at[pl.ds(base, TPW)])


def _run_gather(ys, pos0, pos1):
    mesh = plsc.VectorSubcoreMesh(core_axis_name="c", subcore_axis_name="s")
    f = functools.partial(
        pl.kernel,
        out_type=(
            jax.ShapeDtypeStruct((T, H2), jnp.int32),
            jax.ShapeDtypeStruct((T, H2), jnp.int32),
        ),
        mesh=mesh,
        scratch_types=[
            pltpu.VMEM((TPW, H2), jnp.int32),
            pltpu.VMEM((TPW, H2), jnp.int32),
            pltpu.VMEM((TPW,), jnp.int32),
            pltpu.VMEM((TPW,), jnp.int32),
            pltpu.SemaphoreType.DMA,
            pltpu.SemaphoreType.DMA,
        ],
    )(_gather_body)
    return f(ys, pos0, pos1)


def _shared_body(x_ref, wgs_ref, wus_ref, wds_ref, y0_ref, y1_ref,
                 w0_ref, w1_ref, out_ref):
    xb = x_ref[...].astype(jnp.bfloat16)
    hg = jnp.dot(xb, wgs_ref[...], preferred_element_type=jnp.float32)
    hu = jnp.dot(xb, wus_ref[...], preferred_element_type=jnp.float32)
    mid = (jax.nn.silu(hg) * hu).astype(jnp.bfloat16)
    sh = jnp.dot(mid, wds_ref[...], preferred_element_type=jnp.float32)
    w0 = w0_ref[...][:, 0:1]
    w1 = w1_ref[...][:, 0:1]
    y0a, y0b = _unpack(y0_ref[...])
    y1a, y1b = _unpack(y1_ref[...])
    out_ref[:, :H2] = sh[:, :H2] + w0 * y0a + w1 * y1a
    out_ref[:, H2:] = sh[:, H2:] + w0 * y0b + w1 * y1b


def _run_shared_combine(x, Wg_s, Wu_s, Wd_s, y0g, y1g, w0, w1):
    return pl.pallas_call(
        _shared_body,
        grid=(T // TBLK,),
        out_shape=jax.ShapeDtypeStruct((T, H), jnp.float32),
        in_specs=[
            pl.BlockSpec((TBLK, H), lambda t: (t, 0)),
            pl.BlockSpec((H, SI), lambda t: (0, 0)),
            pl.BlockSpec((H, SI), lambda t: (0, 0)),
            pl.BlockSpec((SI, H), lambda t: (0, 0)),
            pl.BlockSpec((TBLK, H2), lambda t: (t, 0)),
            pl.BlockSpec((TBLK, H2), lambda t: (t, 0)),
            pl.BlockSpec((TBLK, 16), lambda t: (t, 0)),
            pl.BlockSpec((TBLK, 16), lambda t: (t, 0)),
        ],
        out_specs=pl.BlockSpec((TBLK, H), lambda t: (t, 0)),
    )(x, Wg_s, Wu_s, Wd_s, y0g, y1g, w0, w1)


def kernel(hidden_states, W_gate, Wg_s, Wu_s, Wd_s, Wg, Wu, Wd, expert_bias):
    b, s, h = hidden_states.shape
    x = hidden_states.reshape(T, H)

    pos0, pos1, w0, w1, meta, xpk = _run_router(x, W_gate, expert_bias)
    pos0 = pos0.reshape(T)
    pos1 = pos1.reshape(T)
    meta1d = meta.reshape(MW)

    xs = _run_dispatch(xpk, pos0, pos1)
    ys = _run_grouped(meta1d, xs, Wg.astype(jnp.bfloat16),
                      Wu.astype(jnp.bfloat16), Wd.astype(jnp.bfloat16))
    y0g, y1g = _run_gather(ys, pos0, pos1)
    out = _run_shared_combine(x, Wg_s.astype(jnp.bfloat16),
                              Wu_s.astype(jnp.bfloat16),
                              Wd_s.astype(jnp.bfloat16), y0g, y1g, w0, w1)
    return out.reshape(b, s, h)


# speedup vs baseline: 4.6289x; 1.1280x over previous
import functools

import jax
import jax.numpy as jnp
from jax import lax
from jax.experimental import pallas as pl
from jax.experimental.pallas import tpu as pltpu
from jax.experimental.pallas import tpu_sc as plsc

T = 2048
H = 1024
H2 = H // 2
E = 8
K = 2
INTER = 512
SI = 1024
BM = 256
NBLK = T * K // BM + E
NR = NBLK * BM
MW = NBLK + 1
NC = 2
NS = 16
NW = NC * NS
TPW = T // NW
TBLK = 512

def _pack(lo_f32, hi_f32):
    mask = jnp.full(lo_f32.shape, 0xFFFF0000, jnp.uint32)
    lo = pltpu.bitcast(lo_f32.astype(jnp.bfloat16).astype(jnp.float32),
                       jnp.uint32) >> 16
    hi = pltpu.bitcast(hi_f32.astype(jnp.bfloat16).astype(jnp.float32),
                       jnp.uint32) & mask
    return pltpu.bitcast(lo | hi, jnp.int32)


def _unpack(word_i32):
    mask = jnp.full(word_i32.shape, 0xFFFF0000, jnp.uint32)
    w = pltpu.bitcast(word_i32, jnp.uint32)
    lo = pltpu.bitcast(w << 16, jnp.float32)
    hi = pltpu.bitcast(w & mask, jnp.float32)
    return lo, hi


def _router_body(x_ref, wg_ref, b_ref, pos0_ref, pos1_ref, w0_ref, w1_ref,
                 meta_ref, xpk_ref):
    x = x_ref[...]
    xpk_ref[...] = _pack(x[:, :H2], x[:, H2:])
    scores = jax.nn.sigmoid(
        jnp.dot(x, wg_ref[...], preferred_element_type=jnp.float32))
    biased = scores + b_ref[...]
    iota = lax.broadcasted_iota(jnp.int32, (T, E), 1)
    m0 = jnp.max(biased, axis=1, keepdims=True)
    sel0 = jnp.min(jnp.where(biased >= m0, iota, E), axis=1, keepdims=True)
    neg = jnp.where(iota == sel0, -jnp.inf, biased)
    m1 = jnp.max(neg, axis=1, keepdims=True)
    sel1 = jnp.min(jnp.where(neg >= m1, iota, E), axis=1, keepdims=True)
    s0 = jnp.sum(jnp.where(iota == sel0, scores, 0.0), axis=1, keepdims=True)
    s1 = jnp.sum(jnp.where(iota == sel1, scores, 0.0), axis=1, keepdims=True)
    denom = s0 + s1 + 1e-20
    w0_ref[...] = jnp.broadcast_to(s0 / denom, (T, 16))
    w1_ref[...] = jnp.broadcast_to(s1 / denom, (T, 16))

    csum = jnp.logical_or(iota == sel0, iota == sel1).astype(jnp.float32)
    shift = 1
    while shift < T:
        shifted = jnp.concatenate(
            [jnp.zeros((shift, E), jnp.float32), csum[:T - shift]], axis=0)
        csum = csum + shifted
        shift *= 2
    counts = csum[T - 1:T, :]
    cnt_pad = jnp.floor((counts + (BM - 1)) / BM) * BM
    er = lax.broadcasted_iota(jnp.int32, (E, E), 0)
    ec = lax.broadcasted_iota(jnp.int32, (E, E), 1)
    upper = (er < ec).astype(jnp.float32)
    offs = jnp.dot(cnt_pad, upper, preferred_element_type=jnp.float32)

    posf0 = jnp.sum(jnp.where(iota == sel0, offs + csum - 1.0, 0.0),
                    axis=1, keepdims=True)
    posf1 = jnp.sum(jnp.where(iota == sel1, offs + csum - 1.0, 0.0),
                    axis=1, keepdims=True)
    pos0_ref[...] = posf0.astype(jnp.int32)
    pos1_ref[...] = posf1.astype(jnp.int32)

    evec = lax.broadcasted_iota(jnp.int32, (1, E), 1)
    off_s = [jnp.sum(jnp.where(evec == e, offs, 0.0)) for e in range(E)]
    cnt_s = [jnp.sum(jnp.where(evec == e, cnt_pad, 0.0)) for e in range(E)]
    nb_total = ((off_s[E - 1] + cnt_s[E - 1]) / BM).astype(jnp.int32)
    bio = lax.broadcasted_iota(jnp.int32, (1, MW), 1)
    bvals = (bio - 1) * BM
    be = sum(((bvals.astype(jnp.float32) >= off_s[e]).astype(jnp.int32))
             for e in range(E)) - 1
    meta_ref[...] = jnp.where(bio == 0, nb_total, be)


def _run_router(x, W_gate, expert_bias):
    return pl.pallas_call(
        _router_body,
        out_shape=(
            jax.ShapeDtypeStruct((T, 1), jnp.int32),
            jax.ShapeDtypeStruct((T, 1), jnp.int32),
            jax.ShapeDtypeStruct((T, 16), jnp.float32),
            jax.ShapeDtypeStruct((T, 16), jnp.float32),
            jax.ShapeDtypeStruct((1, MW), jnp.int32),
            jax.ShapeDtypeStruct((T, H2), jnp.int32),
        ),
        in_specs=[
            pl.BlockSpec((T, H), lambda: (0, 0)),
            pl.BlockSpec((H, E), lambda: (0, 0)),
            pl.BlockSpec((1, E), lambda: (0, 0)),
        ],
        out_specs=(
            pl.BlockSpec((T, 1), lambda: (0, 0)),
            pl.BlockSpec((T, 1), lambda: (0, 0)),
            pl.BlockSpec((T, 16), lambda: (0, 0)),
            pl.BlockSpec((T, 16), lambda: (0, 0)),
            pl.BlockSpec((1, MW), lambda: (0, 0)),
            pl.BlockSpec((T, H2), lambda: (0, 0)),
        ),
    )(x, W_gate, expert_bias.reshape(1, E))


def _dispatch_body(xpk_hbm, p0_hbm, p1_hbm, xs_hbm, xrows, p0v, p1v,
                   sem0, sem1):
    wid = lax.axis_index("s") * NC + lax.axis_index("c")
    base = wid * TPW
    pltpu.sync_copy(xpk_hbm.at[pl.ds(base, TPW)], xrows)
    pltpu.sync_copy(p0_hbm.at[pl.ds(base, TPW)], p0v)
    pltpu.sync_copy(p1_hbm.at[pl.ds(base, TPW)], p1v)
    a = pltpu.async_copy(xrows, xs_hbm.at[p0v], sem0)
    b = pltpu.async_copy(xrows, xs_hbm.at[p1v], sem1)
    a.wait()
    b.wait()


def _run_dispatch(xpk, pos0, pos1):
    mesh = plsc.VectorSubcoreMesh(core_axis_name="c", subcore_axis_name="s")
    f = functools.partial(
        pl.kernel,
        out_type=jax.ShapeDtypeStruct((NR, H2), jnp.int32),
        mesh=mesh,
        scratch_types=[
            pltpu.VMEM((TPW, H2), jnp.int32),
            pltpu.VMEM((TPW,), jnp.int32),
            pltpu.VMEM((TPW,), jnp.int32),
            pltpu.SemaphoreType.DMA,
            pltpu.SemaphoreType.DMA,
        ],
    )(_dispatch_body)
    return f(xpk, pos0, pos1)


def _grouped_body(m_ref, xs_ref, wg_ref, wu_ref, wd_ref, ys_ref,
                  wgc, wuc, wdc):
    b = pl.program_id(0)
    active = b < m_ref[0]
    changed = jnp.logical_or(b == 0, m_ref[b + 1] != m_ref[b])

    @pl.when(jnp.logical_and(active, changed))
    def _():
        wgc[...] = wg_ref[0].astype(jnp.bfloat16)
        wuc[...] = wu_ref[0].astype(jnp.bfloat16)
        wdc[...] = wd_ref[0].astype(jnp.bfloat16)

    @pl.when(active)
    def _():
        xa, xb_ = _unpack(xs_ref[...])
        xa = xa.astype(jnp.bfloat16)
        xb_ = xb_.astype(jnp.bfloat16)
        hg = (jnp.dot(xa, wgc[:H2, :], preferred_element_type=jnp.float32)
              + jnp.dot(xb_, wgc[H2:, :], preferred_element_type=jnp.float32))
        hu = (jnp.dot(xa, wuc[:H2, :], preferred_element_type=jnp.float32)
              + jnp.dot(xb_, wuc[H2:, :], preferred_element_type=jnp.float32))
        mid = (jax.nn.silu(hg) * hu).astype(jnp.bfloat16)
        ys = jnp.dot(mid, wdc[...], preferred_element_type=jnp.float32)
        ys_ref[...] = _pack(ys[:, :H2], ys[:, H2:])


def _run_grouped(meta1d, xs, Wg, Wu, Wd):
    grid_spec = pltpu.PrefetchScalarGridSpec(
        num_scalar_prefetch=1,
        grid=(NBLK,),
        in_specs=[
            pl.BlockSpec((BM, H2), lambda b, m: (b, 0)),
            pl.BlockSpec((1, H, INTER), lambda b, m: (m[b + 1], 0, 0)),
            pl.BlockSpec((1, H, INTER), lambda b, m: (m[b + 1], 0, 0)),
            pl.BlockSpec((1, INTER, H), lambda b, m: (m[b + 1], 0, 0)),
        ],
        out_specs=pl.BlockSpec((BM, H2), lambda b, m: (b, 0)),
        scratch_shapes=[
            pltpu.VMEM((H, INTER), jnp.bfloat16),
            pltpu.VMEM((H, INTER), jnp.bfloat16),
            pltpu.VMEM((INTER, H), jnp.bfloat16),
        ],
    )
    return pl.pallas_call(
        _grouped_body,
        grid_spec=grid_spec,
        out_shape=jax.ShapeDtypeStruct((NR, H2), jnp.int32),
    )(meta1d, xs, Wg, Wu, Wd)


def _gather_body(ys_hbm, p0_hbm, p1_hbm, y0g_hbm, y1g_hbm,
                 y0b, y1b, p0v, p1v, sem0, sem1):
    wid = lax.axis_index("s") * NC + lax.axis_index("c")
    base = wid * TPW
    pltpu.sync_copy(p0_hbm.at[pl.ds(base, TPW)], p0v)
    pltpu.sync_copy(p1_hbm.at[pl.ds(base, TPW)], p1v)
    a = pltpu.async_copy(ys_hbm.at[p0v], y0b, sem0)
    b = pltpu.async_copy(ys_hbm.at[p1v], y1b, sem1)
    a.wait()
    b.wait()
    pltpu.sync_copy(y0b, y0g_hbm.at[pl.ds(base, TPW)])
    pltpu.sync_copy(y1b, y1g_hbm.at[pl.ds(base, TPW)])


def _run_gather(ys, pos0, pos1):
    mesh = plsc.VectorSubcoreMesh(core_axis_name="c", subcore_axis_name="s")
    f = functools.partial(
        pl.kernel,
        out_type=(
            jax.ShapeDtypeStruct((T, H2), jnp.int32),
            jax.ShapeDtypeStruct((T, H2), jnp.int32),
        ),
        mesh=mesh,
        scratch_types=[
            pltpu.VMEM((TPW, H2), jnp.int32),
            pltpu.VMEM((TPW, H2), jnp.int32),
            pltpu.VMEM((TPW,), jnp.int32),
            pltpu.VMEM((TPW,), jnp.int32),
            pltpu.SemaphoreType.DMA,
            pltpu.SemaphoreType.DMA,
        ],
    )(_gather_body)
    return f(ys, pos0, pos1)


def _shared_body(x_ref, wgs_ref, wus_ref, wds_ref, y0_ref, y1_ref,
                 w0_ref, w1_ref, out_ref, wgsc, wusc, wdsc):
    @pl.when(pl.program_id(0) == 0)
    def _():
        wgsc[...] = wgs_ref[...].astype(jnp.bfloat16)
        wusc[...] = wus_ref[...].astype(jnp.bfloat16)
        wdsc[...] = wds_ref[...].astype(jnp.bfloat16)

    xb = x_ref[...].astype(jnp.bfloat16)
    hg = jnp.dot(xb, wgsc[...], preferred_element_type=jnp.float32)
    hu = jnp.dot(xb, wusc[...], preferred_element_type=jnp.float32)
    mid = (jax.nn.silu(hg) * hu).astype(jnp.bfloat16)
    sh = jnp.dot(mid, wdsc[...], preferred_element_type=jnp.float32)
    w0 = w0_ref[...][:, 0:1]
    w1 = w1_ref[...][:, 0:1]
    y0a, y0b = _unpack(y0_ref[...])
    y1a, y1b = _unpack(y1_ref[...])
    out_ref[:, :H2] = sh[:, :H2] + w0 * y0a + w1 * y1a
    out_ref[:, H2:] = sh[:, H2:] + w0 * y0b + w1 * y1b


def _run_shared_combine(x, Wg_s, Wu_s, Wd_s, y0g, y1g, w0, w1):
    return pl.pallas_call(
        _shared_body,
        grid=(T // TBLK,),
        out_shape=jax.ShapeDtypeStruct((T, H), jnp.float32),
        in_specs=[
            pl.BlockSpec((TBLK, H), lambda t: (t, 0)),
            pl.BlockSpec((H, SI), lambda t: (0, 0)),
            pl.BlockSpec((H, SI), lambda t: (0, 0)),
            pl.BlockSpec((SI, H), lambda t: (0, 0)),
            pl.BlockSpec((TBLK, H2), lambda t: (t, 0)),
            pl.BlockSpec((TBLK, H2), lambda t: (t, 0)),
            pl.BlockSpec((TBLK, 16), lambda t: (t, 0)),
            pl.BlockSpec((TBLK, 16), lambda t: (t, 0)),
        ],
        out_specs=pl.BlockSpec((TBLK, H), lambda t: (t, 0)),
        scratch_shapes=[
            pltpu.VMEM((H, SI), jnp.bfloat16),
            pltpu.VMEM((H, SI), jnp.bfloat16),
            pltpu.VMEM((SI, H), jnp.bfloat16),
        ],
    )(x, Wg_s, Wu_s, Wd_s, y0g, y1g, w0, w1)


def kernel(hidden_states, W_gate, Wg_s, Wu_s, Wd_s, Wg, Wu, Wd, expert_bias):
    b, s, h = hidden_states.shape
    x = hidden_states.reshape(T, H)

    pos0, pos1, w0, w1, meta, xpk = _run_router(x, W_gate, expert_bias)
    pos0 = pos0.reshape(T)
    pos1 = pos1.reshape(T)
    meta1d = meta.reshape(MW)

    xs = _run_dispatch(xpk, pos0, pos1)
    ys = _run_grouped(meta1d, xs, Wg, Wu, Wd)
    y0g, y1g = _run_gather(ys, pos0, pos1)
    out = _run_shared_combine(x, Wg_s, Wu_s, Wd_s, y0g, y1g, w0, w1)
    return out.reshape(b, s, h)


# speedup vs baseline: 4.6310x; 1.0004x over previous
import functools

import jax
import jax.numpy as jnp
from jax import lax
from jax.experimental import pallas as pl
from jax.experimental.pallas import tpu as pltpu
from jax.experimental.pallas import tpu_sc as plsc

T = 2048
H = 1024
H2 = H // 2
E = 8
K = 2
INTER = 512
SI = 1024
BM = 256
NBLK = T * K // BM + E
NR = NBLK * BM
MW = NBLK + 1
NC = 2
NS = 16
NW = NC * NS
TPW = T // NW
TBLK = 512

def _pack(lo_f32, hi_f32):
    mask = jnp.full(lo_f32.shape, 0xFFFF0000, jnp.uint32)
    lo = pltpu.bitcast(lo_f32.astype(jnp.bfloat16).astype(jnp.float32),
                       jnp.uint32) >> 16
    hi = pltpu.bitcast(hi_f32.astype(jnp.bfloat16).astype(jnp.float32),
                       jnp.uint32) & mask
    return pltpu.bitcast(lo | hi, jnp.int32)


def _unpack(word_i32):
    mask = jnp.full(word_i32.shape, 0xFFFF0000, jnp.uint32)
    w = pltpu.bitcast(word_i32, jnp.uint32)
    lo = pltpu.bitcast(w << 16, jnp.float32)
    hi = pltpu.bitcast(w & mask, jnp.float32)
    return lo, hi


def _router_body(x_ref, wg_ref, b_ref, pos0_ref, pos1_ref, w0_ref, w1_ref,
                 meta_ref, xpk_ref):
    x = x_ref[...]
    xpk_ref[...] = _pack(x[:, :H2], x[:, H2:])
    scores = jax.nn.sigmoid(
        jnp.dot(x, wg_ref[...], preferred_element_type=jnp.float32))
    biased = scores + b_ref[...]
    iota = lax.broadcasted_iota(jnp.int32, (T, E), 1)
    m0 = jnp.max(biased, axis=1, keepdims=True)
    sel0 = jnp.min(jnp.where(biased >= m0, iota, E), axis=1, keepdims=True)
    neg = jnp.where(iota == sel0, -jnp.inf, biased)
    m1 = jnp.max(neg, axis=1, keepdims=True)
    sel1 = jnp.min(jnp.where(neg >= m1, iota, E), axis=1, keepdims=True)
    s0 = jnp.sum(jnp.where(iota == sel0, scores, 0.0), axis=1, keepdims=True)
    s1 = jnp.sum(jnp.where(iota == sel1, scores, 0.0), axis=1, keepdims=True)
    denom = s0 + s1 + 1e-20
    w0_ref[...] = jnp.broadcast_to(s0 / denom, (T, 16))
    w1_ref[...] = jnp.broadcast_to(s1 / denom, (T, 16))

    csum = jnp.logical_or(iota == sel0, iota == sel1).astype(jnp.float32)
    shift = 1
    while shift < T:
        shifted = jnp.concatenate(
            [jnp.zeros((shift, E), jnp.float32), csum[:T - shift]], axis=0)
        csum = csum + shifted
        shift *= 2
    counts = csum[T - 1:T, :]
    cnt_pad = jnp.floor((counts + (BM - 1)) / BM) * BM
    er = lax.broadcasted_iota(jnp.int32, (E, E), 0)
    ec = lax.broadcasted_iota(jnp.int32, (E, E), 1)
    upper = (er < ec).astype(jnp.float32)
    offs = jnp.dot(cnt_pad, upper, preferred_element_type=jnp.float32)

    posf0 = jnp.sum(jnp.where(iota == sel0, offs + csum - 1.0, 0.0),
                    axis=1, keepdims=True)
    posf1 = jnp.sum(jnp.where(iota == sel1, offs + csum - 1.0, 0.0),
                    axis=1, keepdims=True)
    pos0_ref[...] = posf0.astype(jnp.int32)
    pos1_ref[...] = posf1.astype(jnp.int32)

    evec = lax.broadcasted_iota(jnp.int32, (1, E), 1)
    off_s = [jnp.sum(jnp.where(evec == e, offs, 0.0)) for e in range(E)]
    cnt_s = [jnp.sum(jnp.where(evec == e, cnt_pad, 0.0)) for e in range(E)]
    nb_total = ((off_s[E - 1] + cnt_s[E - 1]) / BM).astype(jnp.int32)
    bio = lax.broadcasted_iota(jnp.int32, (1, MW), 1)
    bvals = (bio - 1) * BM
    be = sum(((bvals.astype(jnp.float32) >= off_s[e]).astype(jnp.int32))
             for e in range(E)) - 1
    meta_ref[...] = jnp.where(bio == 0, nb_total, be)


def _run_router(x, W_gate, expert_bias):
    return pl.pallas_call(
        _router_body,
        out_shape=(
            jax.ShapeDtypeStruct((T, 1), jnp.int32),
            jax.ShapeDtypeStruct((T, 1), jnp.int32),
            jax.ShapeDtypeStruct((T, 16), jnp.float32),
            jax.ShapeDtypeStruct((T, 16), jnp.float32),
            jax.ShapeDtypeStruct((1, MW), jnp.int32),
            jax.ShapeDtypeStruct((T, H2), jnp.int32),
        ),
        in_specs=[
            pl.BlockSpec((T, H), lambda: (0, 0)),
            pl.BlockSpec((H, E), lambda: (0, 0)),
            pl.BlockSpec((1, E), lambda: (0, 0)),
        ],
        out_specs=(
            pl.BlockSpec((T, 1), lambda: (0, 0)),
            pl.BlockSpec((T, 1), lambda: (0, 0)),
            pl.BlockSpec((T, 16), lambda: (0, 0)),
            pl.BlockSpec((T, 16), lambda: (0, 0)),
            pl.BlockSpec((1, MW), lambda: (0, 0)),
            pl.BlockSpec((T, H2), lambda: (0, 0)),
        ),
    )(x, W_gate, expert_bias.reshape(1, E))


def _dispatch_body(xpk_hbm, p0_hbm, p1_hbm, xs_hbm, xrows, p0v, p1v,
                   sem0, sem1, sem2):
    wid = lax.axis_index("s") * NC + lax.axis_index("c")
    base = wid * TPW
    c0 = pltpu.async_copy(xpk_hbm.at[pl.ds(base, TPW)], xrows, sem0)
    c1 = pltpu.async_copy(p0_hbm.at[pl.ds(base, TPW)], p0v, sem1)
    c2 = pltpu.async_copy(p1_hbm.at[pl.ds(base, TPW)], p1v, sem2)
    c0.wait()
    c1.wait()
    c2.wait()
    a = pltpu.async_copy(xrows, xs_hbm.at[p0v], sem0)
    b = pltpu.async_copy(xrows, xs_hbm.at[p1v], sem1)
    a.wait()
    b.wait()


def _run_dispatch(xpk, pos0, pos1):
    mesh = plsc.VectorSubcoreMesh(core_axis_name="c", subcore_axis_name="s")
    f = functools.partial(
        pl.kernel,
        out_type=jax.ShapeDtypeStruct((NR, H2), jnp.int32),
        mesh=mesh,
        scratch_types=[
            pltpu.VMEM((TPW, H2), jnp.int32),
            pltpu.VMEM((TPW,), jnp.int32),
            pltpu.VMEM((TPW,), jnp.int32),
            pltpu.SemaphoreType.DMA,
            pltpu.SemaphoreType.DMA,
            pltpu.SemaphoreType.DMA,
        ],
    )(_dispatch_body)
    return f(xpk, pos0, pos1)


def _grouped_body(m_ref, xs_ref, wg_ref, wu_ref, wd_ref, ys_ref,
                  wgc, wuc, wdc):
    b = pl.program_id(0)
    active = b < m_ref[0]
    changed = jnp.logical_or(b == 0, m_ref[b + 1] != m_ref[b])

    @pl.when(jnp.logical_and(active, changed))
    def _():
        wgc[...] = wg_ref[0].astype(jnp.bfloat16)
        wuc[...] = wu_ref[0].astype(jnp.bfloat16)
        wdc[...] = wd_ref[0].astype(jnp.bfloat16)

    @pl.when(active)
    def _():
        xa, xb_ = _unpack(xs_ref[...])
        xa = xa.astype(jnp.bfloat16)
        xb_ = xb_.astype(jnp.bfloat16)
        hg = (jnp.dot(xa, wgc[:H2, :], preferred_element_type=jnp.float32)
              + jnp.dot(xb_, wgc[H2:, :], preferred_element_type=jnp.float32))
        hu = (jnp.dot(xa, wuc[:H2, :], preferred_element_type=jnp.float32)
              + jnp.dot(xb_, wuc[H2:, :], preferred_element_type=jnp.float32))
        mid = (jax.nn.silu(hg) * hu).astype(jnp.bfloat16)
        ys = jnp.dot(mid, wdc[...], preferred_element_type=jnp.float32)
        ys_ref[...] = _pack(ys[:, :H2], ys[:, H2:])


def _run_grouped(meta1d, xs, Wg, Wu, Wd):
    grid_spec = pltpu.PrefetchScalarGridSpec(
        num_scalar_prefetch=1,
        grid=(NBLK,),
        in_specs=[
            pl.BlockSpec((BM, H2), lambda b, m: (b, 0)),
            pl.BlockSpec((1, H, INTER), lambda b, m: (m[b + 1], 0, 0)),
            pl.BlockSpec((1, H, INTER), lambda b, m: (m[b + 1], 0, 0)),
            pl.BlockSpec((1, INTER, H), lambda b, m: (m[b + 1], 0, 0)),
        ],
        out_specs=pl.BlockSpec((BM, H2), lambda b, m: (b, 0)),
        scratch_shapes=[
            pltpu.VMEM((H, INTER), jnp.bfloat16),
            pltpu.VMEM((H, INTER), jnp.bfloat16),
            pltpu.VMEM((INTER, H), jnp.bfloat16),
        ],
    )
    return pl.pallas_call(
        _grouped_body,
        grid_spec=grid_spec,
        out_shape=jax.ShapeDtypeStruct((NR, H2), jnp.int32),
    )(meta1d, xs, Wg, Wu, Wd)


def _gather_body(ys_hbm, p0_hbm, p1_hbm, y0g_hbm, y1g_hbm,
                 y0b, y1b, p0v, p1v, sem0, sem1):
    wid = lax.axis_index("s") * NC + lax.axis_index("c")
    base = wid * TPW
    c0 = pltpu.async_copy(p0_hbm.at[pl.ds(base, TPW)], p0v, sem0)
    c1 = pltpu.async_copy(p1_hbm.at[pl.ds(base, TPW)], p1v, sem1)
    c0.wait()
    c1.wait()
    a = pltpu.async_copy(ys_hbm.at[p0v], y0b, sem0)
    b = pltpu.async_copy(ys_hbm.at[p1v], y1b, sem1)
    a.wait()
    c2 = pltpu.async_copy(y0b, y0g_hbm.at[pl.ds(base, TPW)], sem0)
    b.wait()
    c3 = pltpu.async_copy(y1b, y1g_hbm.at[pl.ds(base, TPW)], sem1)
    c2.wait()
    c3.wait()


def _run_gather(ys, pos0, pos1):
    mesh = plsc.VectorSubcoreMesh(core_axis_name="c", subcore_axis_name="s")
    f = functools.partial(
        pl.kernel,
        out_type=(
            jax.ShapeDtypeStruct((T, H2), jnp.int32),
            jax.ShapeDtypeStruct((T, H2), jnp.int32),
        ),
        mesh=mesh,
        scratch_types=[
            pltpu.VMEM((TPW, H2), jnp.int32),
            pltpu.VMEM((TPW, H2), jnp.int32),
            pltpu.VMEM((TPW,), jnp.int32),
            pltpu.VMEM((TPW,), jnp.int32),
            pltpu.SemaphoreType.DMA,
            pltpu.SemaphoreType.DMA,
        ],
    )(_gather_body)
    return f(ys, pos0, pos1)


def _shared_body(x_ref, wgs_ref, wus_ref, wds_ref, out_ref, wgsc, wusc, wdsc):
    @pl.when(pl.program_id(0) == 0)
    def _():
        wgsc[...] = wgs_ref[...].astype(jnp.bfloat16)
        wusc[...] = wus_ref[...].astype(jnp.bfloat16)
        wdsc[...] = wds_ref[...].astype(jnp.bfloat16)

    xb = x_ref[...].astype(jnp.bfloat16)
    hg = jnp.dot(xb, wgsc[...], preferred_element_type=jnp.float32)
    hu = jnp.dot(xb, wusc[...], preferred_element_type=jnp.float32)
    mid = (jax.nn.silu(hg) * hu).astype(jnp.bfloat16)
    out_ref[...] = jnp.dot(mid, wdsc[...], preferred_element_type=jnp.float32)


def _run_shared(x, Wg_s, Wu_s, Wd_s):
    return pl.pallas_call(
        _shared_body,
        grid=(T // TBLK,),
        out_shape=jax.ShapeDtypeStruct((T, H), jnp.float32),
        in_specs=[
            pl.BlockSpec((TBLK, H), lambda t: (t, 0)),
            pl.BlockSpec((H, SI), lambda t: (0, 0)),
            pl.BlockSpec((H, SI), lambda t: (0, 0)),
            pl.BlockSpec((SI, H), lambda t: (0, 0)),
        ],
        out_specs=pl.BlockSpec((TBLK, H), lambda t: (t, 0)),
        scratch_shapes=[
            pltpu.VMEM((H, SI), jnp.bfloat16),
            pltpu.VMEM((H, SI), jnp.bfloat16),
            pltpu.VMEM((SI, H), jnp.bfloat16),
        ],
    )(x, Wg_s, Wu_s, Wd_s)


def _combine_body(sh_ref, y0_ref, y1_ref, w0_ref, w1_ref, out_ref):
    sh = sh_ref[...]
    w0 = w0_ref[...][:, 0:1]
    w1 = w1_ref[...][:, 0:1]
    y0a, y0b = _unpack(y0_ref[...])
    y1a, y1b = _unpack(y1_ref[...])
    out_ref[:, :H2] = sh[:, :H2] + w0 * y0a + w1 * y1a
    out_ref[:, H2:] = sh[:, H2:] + w0 * y0b + w1 * y1b


def _run_combine(sh, y0g, y1g, w0, w1):
    return pl.pallas_call(
        _combine_body,
        grid=(T // TBLK,),
        out_shape=jax.ShapeDtypeStruct((T, H), jnp.float32),
        in_specs=[
            pl.BlockSpec((TBLK, H), lambda t: (t, 0)),
            pl.BlockSpec((TBLK, H2), lambda t: (t, 0)),
            pl.BlockSpec((TBLK, H2), lambda t: (t, 0)),
            pl.BlockSpec((TBLK, 16), lambda t: (t, 0)),
            pl.BlockSpec((TBLK, 16), lambda t: (t, 0)),
        ],
        out_specs=pl.BlockSpec((TBLK, H), lambda t: (t, 0)),
    )(sh, y0g, y1g, w0, w1)


def kernel(hidden_states, W_gate, Wg_s, Wu_s, Wd_s, Wg, Wu, Wd, expert_bias):
    b, s, h = hidden_states.shape
    x = hidden_states.reshape(T, H)

    pos0, pos1, w0, w1, meta, xpk = _run_router(x, W_gate, expert_bias)
    pos0 = pos0.reshape(T)
    pos1 = pos1.reshape(T)
    meta1d = meta.reshape(MW)

    sh = _run_shared(x, Wg_s, Wu_s, Wd_s)
    xs = _run_dispatch(xpk, pos0, pos1)
    ys = _run_grouped(meta1d, xs, Wg, Wu, Wd)
    y0g, y1g = _run_gather(ys, pos0, pos1)
    out = _run_combine(sh, y0g, y1g, w0, w1)
    return out.reshape(b, s, h)


# speedup vs baseline: 4.7324x; 1.0219x over previous
import functools

import jax
import jax.numpy as jnp
from jax import lax
from jax.experimental import pallas as pl
from jax.experimental.pallas import tpu as pltpu
from jax.experimental.pallas import tpu_sc as plsc

T = 2048
H = 1024
H2 = H // 2
E = 8
K = 2
INTER = 512
SI = 1024
BM = 256
NBLK = T * K // BM + E
NR = NBLK * BM
MW = NBLK + 1
NC = 2
NS = 16
NW = NC * NS
TPW = T // NW
TBLK = 512

def _pack(lo_f32, hi_f32):
    mask = jnp.full(lo_f32.shape, 0xFFFF0000, jnp.uint32)
    lo = pltpu.bitcast(lo_f32.astype(jnp.bfloat16).astype(jnp.float32),
                       jnp.uint32) >> 16
    hi = pltpu.bitcast(hi_f32.astype(jnp.bfloat16).astype(jnp.float32),
                       jnp.uint32) & mask
    return pltpu.bitcast(lo | hi, jnp.int32)


def _unpack(word_i32):
    mask = jnp.full(word_i32.shape, 0xFFFF0000, jnp.uint32)
    w = pltpu.bitcast(word_i32, jnp.uint32)
    lo = pltpu.bitcast(w << 16, jnp.float32)
    hi = pltpu.bitcast(w & mask, jnp.float32)
    return lo, hi


def _router_body(x_ref, wg_ref, b_ref, pos0_ref, pos1_ref, w0_ref, w1_ref,
                 meta_ref, xpk_ref):
    x = x_ref[...]
    xpk_ref[...] = _pack(x[:, :H2], x[:, H2:])
    scores = jax.nn.sigmoid(
        jnp.dot(x, wg_ref[...], preferred_element_type=jnp.float32))
    biased = scores + b_ref[...]
    iota = lax.broadcasted_iota(jnp.int32, (T, E), 1)
    m0 = jnp.max(biased, axis=1, keepdims=True)
    sel0 = jnp.min(jnp.where(biased >= m0, iota, E), axis=1, keepdims=True)
    neg = jnp.where(iota == sel0, -jnp.inf, biased)
    m1 = jnp.max(neg, axis=1, keepdims=True)
    sel1 = jnp.min(jnp.where(neg >= m1, iota, E), axis=1, keepdims=True)
    s0 = jnp.sum(jnp.where(iota == sel0, scores, 0.0), axis=1, keepdims=True)
    s1 = jnp.sum(jnp.where(iota == sel1, scores, 0.0), axis=1, keepdims=True)
    denom = s0 + s1 + 1e-20
    w0_ref[...] = jnp.broadcast_to(s0 / denom, (T, 16))
    w1_ref[...] = jnp.broadcast_to(s1 / denom, (T, 16))

    csum = jnp.logical_or(iota == sel0, iota == sel1).astype(jnp.float32)
    shift = 1
    while shift < T:
        shifted = jnp.concatenate(
            [jnp.zeros((shift, E), jnp.float32), csum[:T - shift]], axis=0)
        csum = csum + shifted
        shift *= 2
    counts = csum[T - 1:T, :]
    cnt_pad = jnp.floor((counts + (BM - 1)) / BM) * BM
    er = lax.broadcasted_iota(jnp.int32, (E, E), 0)
    ec = lax.broadcasted_iota(jnp.int32, (E, E), 1)
    upper = (er < ec).astype(jnp.float32)
    offs = jnp.dot(cnt_pad, upper, preferred_element_type=jnp.float32)

    posf0 = jnp.sum(jnp.where(iota == sel0, offs + csum - 1.0, 0.0),
                    axis=1, keepdims=True)
    posf1 = jnp.sum(jnp.where(iota == sel1, offs + csum - 1.0, 0.0),
                    axis=1, keepdims=True)
    pos0_ref[...] = posf0.astype(jnp.int32)
    pos1_ref[...] = posf1.astype(jnp.int32)

    evec = lax.broadcasted_iota(jnp.int32, (1, E), 1)
    off_s = [jnp.sum(jnp.where(evec == e, offs, 0.0)) for e in range(E)]
    cnt_s = [jnp.sum(jnp.where(evec == e, cnt_pad, 0.0)) for e in range(E)]
    nb_total = ((off_s[E - 1] + cnt_s[E - 1]) / BM).astype(jnp.int32)
    bio = lax.broadcasted_iota(jnp.int32, (1, MW), 1)
    bvals = (bio - 1) * BM
    be = sum(((bvals.astype(jnp.float32) >= off_s[e]).astype(jnp.int32))
             for e in range(E)) - 1
    meta_ref[...] = jnp.where(bio == 0, nb_total, be)


def _run_router(x, W_gate, expert_bias):
    return pl.pallas_call(
        _router_body,
        out_shape=(
            jax.ShapeDtypeStruct((T, 1), jnp.int32),
            jax.ShapeDtypeStruct((T, 1), jnp.int32),
            jax.ShapeDtypeStruct((T, 16), jnp.float32),
            jax.ShapeDtypeStruct((T, 16), jnp.float32),
            jax.ShapeDtypeStruct((1, MW), jnp.int32),
            jax.ShapeDtypeStruct((T, H2), jnp.int32),
        ),
        in_specs=[
            pl.BlockSpec((T, H), lambda: (0, 0)),
            pl.BlockSpec((H, E), lambda: (0, 0)),
            pl.BlockSpec((1, E), lambda: (0, 0)),
        ],
        out_specs=(
            pl.BlockSpec((T, 1), lambda: (0, 0)),
            pl.BlockSpec((T, 1), lambda: (0, 0)),
            pl.BlockSpec((T, 16), lambda: (0, 0)),
            pl.BlockSpec((T, 16), lambda: (0, 0)),
            pl.BlockSpec((1, MW), lambda: (0, 0)),
            pl.BlockSpec((T, H2), lambda: (0, 0)),
        ),
    )(x, W_gate, expert_bias.reshape(1, E))


def _dispatch_body(xpk_hbm, p0_hbm, p1_hbm, xs_hbm, xrows, p0v, p1v,
                   sem0, sem1, sem2):
    wid = lax.axis_index("s") * NC + lax.axis_index("c")
    base = wid * TPW
    c0 = pltpu.async_copy(xpk_hbm.at[pl.ds(base, TPW)], xrows, sem0)
    c1 = pltpu.async_copy(p0_hbm.at[pl.ds(base, TPW)], p0v, sem1)
    c2 = pltpu.async_copy(p1_hbm.at[pl.ds(base, TPW)], p1v, sem2)
    c0.wait()
    c1.wait()
    c2.wait()
    a = pltpu.async_copy(xrows, xs_hbm.at[p0v], sem0)
    b = pltpu.async_copy(xrows, xs_hbm.at[p1v], sem1)
    a.wait()
    b.wait()


def _run_dispatch(xpk, pos0, pos1):
    mesh = plsc.VectorSubcoreMesh(core_axis_name="c", subcore_axis_name="s")
    f = functools.partial(
        pl.kernel,
        out_type=jax.ShapeDtypeStruct((NR, H2), jnp.int32),
        mesh=mesh,
        scratch_types=[
            pltpu.VMEM((TPW, H2), jnp.int32),
            pltpu.VMEM((TPW,), jnp.int32),
            pltpu.VMEM((TPW,), jnp.int32),
            pltpu.SemaphoreType.DMA,
            pltpu.SemaphoreType.DMA,
            pltpu.SemaphoreType.DMA,
        ],
    )(_dispatch_body)
    return f(xpk, pos0, pos1)


def _grouped_body(m_ref, xs_ref, wg_ref, wu_ref, wd_ref, ys_ref):
    b = pl.program_id(0)
    active = b < m_ref[0]

    @pl.when(active)
    def _():
        xa, xb_ = _unpack(xs_ref[...])
        hg = (jnp.dot(xa, wg_ref[0, :H2, :],
                      preferred_element_type=jnp.float32)
              + jnp.dot(xb_, wg_ref[0, H2:, :],
                        preferred_element_type=jnp.float32))
        hu = (jnp.dot(xa, wu_ref[0, :H2, :],
                      preferred_element_type=jnp.float32)
              + jnp.dot(xb_, wu_ref[0, H2:, :],
                        preferred_element_type=jnp.float32))
        mid = jax.nn.silu(hg) * hu
        ys = jnp.dot(mid, wd_ref[0], preferred_element_type=jnp.float32)
        ys_ref[...] = _pack(ys[:, :H2], ys[:, H2:])


def _run_grouped(meta1d, xs, Wg, Wu, Wd):
    grid_spec = pltpu.PrefetchScalarGridSpec(
        num_scalar_prefetch=1,
        grid=(NBLK,),
        in_specs=[
            pl.BlockSpec((BM, H2), lambda b, m: (b, 0)),
            pl.BlockSpec((1, H, INTER), lambda b, m: (m[b + 1], 0, 0)),
            pl.BlockSpec((1, H, INTER), lambda b, m: (m[b + 1], 0, 0)),
            pl.BlockSpec((1, INTER, H), lambda b, m: (m[b + 1], 0, 0)),
        ],
        out_specs=pl.BlockSpec((BM, H2), lambda b, m: (b, 0)),
    )
    return pl.pallas_call(
        _grouped_body,
        grid_spec=grid_spec,
        out_shape=jax.ShapeDtypeStruct((NR, H2), jnp.int32),
    )(meta1d, xs, Wg, Wu, Wd)


def _gather_body(ys_hbm, p0_hbm, p1_hbm, y0g_hbm, y1g_hbm,
                 y0b, y1b, p0v, p1v, sem0, sem1):
    wid = lax.axis_index("s") * NC + lax.axis_index("c")
    base = wid * TPW
    c0 = pltpu.async_copy(p0_hbm.at[pl.ds(base, TPW)], p0v, sem0)
    c1 = pltpu.async_copy(p1_hbm.at[pl.ds(base, TPW)], p1v, sem1)
    c0.wait()
    c1.wait()
    a = pltpu.async_copy(ys_hbm.at[p0v], y0b, sem0)
    b = pltpu.async_copy(ys_hbm.at[p1v], y1b, sem1)
    a.wait()
    c2 = pltpu.async_copy(y0b, y0g_hbm.at[pl.ds(base, TPW)], sem0)
    b.wait()
    c3 = pltpu.async_copy(y1b, y1g_hbm.at[pl.ds(base, TPW)], sem1)
    c2.wait()
    c3.wait()


def _run_gather(ys, pos0, pos1):
    mesh = plsc.VectorSubcoreMesh(core_axis_name="c", subcore_axis_name="s")
    f = functools.partial(
        pl.kernel,
        out_type=(
            jax.ShapeDtypeStruct((T, H2), jnp.int32),
            jax.ShapeDtypeStruct((T, H2), jnp.int32),
        ),
        mesh=mesh,
        scratch_types=[
            pltpu.VMEM((TPW, H2), jnp.int32),
            pltpu.VMEM((TPW, H2), jnp.int32),
            pltpu.VMEM((TPW,), jnp.int32),
            pltpu.VMEM((TPW,), jnp.int32),
            pltpu.SemaphoreType.DMA,
            pltpu.SemaphoreType.DMA,
        ],
    )(_gather_body)
    return f(ys, pos0, pos1)


def _shared_body(x_ref, wgs_ref, wus_ref, wds_ref, out_ref, wgsc, wusc, wdsc):
    @pl.when(pl.program_id(0) == 0)
    def _():
        wgsc[...] = wgs_ref[...].astype(jnp.bfloat16)
        wusc[...] = wus_ref[...].astype(jnp.bfloat16)
        wdsc[...] = wds_ref[...].astype(jnp.bfloat16)

    xb = x_ref[...].astype(jnp.bfloat16)
    hg = jnp.dot(xb, wgsc[...], preferred_element_type=jnp.float32)
    hu = jnp.dot(xb, wusc[...], preferred_element_type=jnp.float32)
    mid = (jax.nn.silu(hg) * hu).astype(jnp.bfloat16)
    out_ref[...] = jnp.dot(mid, wdsc[...], preferred_element_type=jnp.float32)


def _run_shared(x, Wg_s, Wu_s, Wd_s):
    return pl.pallas_call(
        _shared_body,
        grid=(T // TBLK,),
        out_shape=jax.ShapeDtypeStruct((T, H), jnp.float32),
        in_specs=[
            pl.BlockSpec((TBLK, H), lambda t: (t, 0)),
            pl.BlockSpec((H, SI), lambda t: (0, 0)),
            pl.BlockSpec((H, SI), lambda t: (0, 0)),
            pl.BlockSpec((SI, H), lambda t: (0, 0)),
        ],
        out_specs=pl.BlockSpec((TBLK, H), lambda t: (t, 0)),
        scratch_shapes=[
            pltpu.VMEM((H, SI), jnp.bfloat16),
            pltpu.VMEM((H, SI), jnp.bfloat16),
            pltpu.VMEM((SI, H), jnp.bfloat16),
        ],
    )(x, Wg_s, Wu_s, Wd_s)


def _combine_body(sh_ref, y0_ref, y1_ref, w0_ref, w1_ref, out_ref):
    sh = sh_ref[...]
    w0 = w0_ref[...][:, 0:1]
    w1 = w1_ref[...][:, 0:1]
    y0a, y0b = _unpack(y0_ref[...])
    y1a, y1b = _unpack(y1_ref[...])
    out_ref[:, :H2] = sh[:, :H2] + w0 * y0a + w1 * y1a
    out_ref[:, H2:] = sh[:, H2:] + w0 * y0b + w1 * y1b


def _run_combine(sh, y0g, y1g, w0, w1):
    return pl.pallas_call(
        _combine_body,
        grid=(T // TBLK,),
        out_shape=jax.ShapeDtypeStruct((T, H), jnp.float32),
        in_specs=[
            pl.BlockSpec((TBLK, H), lambda t: (t, 0)),
            pl.BlockSpec((TBLK, H2), lambda t: (t, 0)),
            pl.BlockSpec((TBLK, H2), lambda t: (t, 0)),
            pl.BlockSpec((TBLK, 16), lambda t: (t, 0)),
            pl.BlockSpec((TBLK, 16), lambda t: (t, 0)),
        ],
        out_specs=pl.BlockSpec((TBLK, H), lambda t: (t, 0)),
    )(sh, y0g, y1g, w0, w1)


def kernel(hidden_states, W_gate, Wg_s, Wu_s, Wd_s, Wg, Wu, Wd, expert_bias):
    b, s, h = hidden_states.shape
    x = hidden_states.reshape(T, H)

    pos0, pos1, w0, w1, meta, xpk = _run_router(x, W_gate, expert_bias)
    pos0 = pos0.reshape(T)
    pos1 = pos1.reshape(T)
    meta1d = meta.reshape(MW)

    sh = _run_shared(x, Wg_s, Wu_s, Wd_s)
    xs = _run_dispatch(xpk, pos0, pos1)
    ys = _run_grouped(meta1d, xs, Wg, Wu, Wd)
    y0g, y1g = _run_gather(ys, pos0, pos1)
    out = _run_combine(sh, y0g, y1g, w0, w1)
    return out.reshape(b, s, h)


# speedup vs baseline: 5.0159x; 1.0599x over previous
import functools

import jax
import jax.numpy as jnp
from jax import lax
from jax.experimental import pallas as pl
from jax.experimental.pallas import tpu as pltpu
from jax.experimental.pallas import tpu_sc as plsc

T = 2048
H = 1024
H2 = H // 2
E = 8
K = 2
INTER = 512
SI = 1024
BM = 512
NBLK = T * K // BM + E
NR = NBLK * BM
MW = NBLK + 1
NC = 2
NS = 16
NW = NC * NS
TPW = T // NW
TBLK = 512

def _pack(lo_f32, hi_f32):
    mask = jnp.full(lo_f32.shape, 0xFFFF0000, jnp.uint32)
    lo = pltpu.bitcast(lo_f32.astype(jnp.bfloat16).astype(jnp.float32),
                       jnp.uint32) >> 16
    hi = pltpu.bitcast(hi_f32.astype(jnp.bfloat16).astype(jnp.float32),
                       jnp.uint32) & mask
    return pltpu.bitcast(lo | hi, jnp.int32)


def _unpack(word_i32):
    mask = jnp.full(word_i32.shape, 0xFFFF0000, jnp.uint32)
    w = pltpu.bitcast(word_i32, jnp.uint32)
    lo = pltpu.bitcast(w << 16, jnp.float32)
    hi = pltpu.bitcast(w & mask, jnp.float32)
    return lo, hi


def _router_body(x_ref, wg_ref, b_ref, pos0_ref, pos1_ref, w0_ref, w1_ref,
                 meta_ref, xpk_ref):
    x = x_ref[...]
    xpk_ref[...] = _pack(x[:, :H2], x[:, H2:])
    scores = jax.nn.sigmoid(
        jnp.dot(x, wg_ref[...], preferred_element_type=jnp.float32))
    biased = scores + b_ref[...]
    iota = lax.broadcasted_iota(jnp.int32, (T, E), 1)
    m0 = jnp.max(biased, axis=1, keepdims=True)
    sel0 = jnp.min(jnp.where(biased >= m0, iota, E), axis=1, keepdims=True)
    neg = jnp.where(iota == sel0, -jnp.inf, biased)
    m1 = jnp.max(neg, axis=1, keepdims=True)
    sel1 = jnp.min(jnp.where(neg >= m1, iota, E), axis=1, keepdims=True)
    s0 = jnp.sum(jnp.where(iota == sel0, scores, 0.0), axis=1, keepdims=True)
    s1 = jnp.sum(jnp.where(iota == sel1, scores, 0.0), axis=1, keepdims=True)
    denom = s0 + s1 + 1e-20
    w0_ref[...] = jnp.broadcast_to(s0 / denom, (T, 16))
    w1_ref[...] = jnp.broadcast_to(s1 / denom, (T, 16))

    csum = jnp.logical_or(iota == sel0, iota == sel1).astype(jnp.float32)
    shift = 1
    while shift < T:
        shifted = jnp.concatenate(
            [jnp.zeros((shift, E), jnp.float32), csum[:T - shift]], axis=0)
        csum = csum + shifted
        shift *= 2
    counts = csum[T - 1:T, :]
    cnt_pad = jnp.floor((counts + (BM - 1)) / BM) * BM
    er = lax.broadcasted_iota(jnp.int32, (E, E), 0)
    ec = lax.broadcasted_iota(jnp.int32, (E, E), 1)
    upper = (er < ec).astype(jnp.float32)
    offs = jnp.dot(cnt_pad, upper, preferred_element_type=jnp.float32)

    posf0 = jnp.sum(jnp.where(iota == sel0, offs + csum - 1.0, 0.0),
                    axis=1, keepdims=True)
    posf1 = jnp.sum(jnp.where(iota == sel1, offs + csum - 1.0, 0.0),
                    axis=1, keepdims=True)
    pos0_ref[...] = posf0.astype(jnp.int32)
    pos1_ref[...] = posf1.astype(jnp.int32)

    evec = lax.broadcasted_iota(jnp.int32, (1, E), 1)
    off_s = [jnp.sum(jnp.where(evec == e, offs, 0.0)) for e in range(E)]
    cnt_s = [jnp.sum(jnp.where(evec == e, cnt_pad, 0.0)) for e in range(E)]
    nb_total = ((off_s[E - 1] + cnt_s[E - 1]) / BM).astype(jnp.int32)
    bio = lax.broadcasted_iota(jnp.int32, (1, MW), 1)
    bvals = (bio - 1) * BM
    be = sum(((bvals.astype(jnp.float32) >= off_s[e]).astype(jnp.int32))
             for e in range(E)) - 1
    meta_ref[...] = jnp.where(bio == 0, nb_total, be)


def _run_router(x, W_gate, expert_bias):
    return pl.pallas_call(
        _router_body,
        out_shape=(
            jax.ShapeDtypeStruct((T, 1), jnp.int32),
            jax.ShapeDtypeStruct((T, 1), jnp.int32),
            jax.ShapeDtypeStruct((T, 16), jnp.float32),
            jax.ShapeDtypeStruct((T, 16), jnp.float32),
            jax.ShapeDtypeStruct((1, MW), jnp.int32),
            jax.ShapeDtypeStruct((T, H2), jnp.int32),
        ),
        in_specs=[
            pl.BlockSpec((T, H), lambda: (0, 0)),
            pl.BlockSpec((H, E), lambda: (0, 0)),
            pl.BlockSpec((1, E), lambda: (0, 0)),
        ],
        out_specs=(
            pl.BlockSpec((T, 1), lambda: (0, 0)),
            pl.BlockSpec((T, 1), lambda: (0, 0)),
            pl.BlockSpec((T, 16), lambda: (0, 0)),
            pl.BlockSpec((T, 16), lambda: (0, 0)),
            pl.BlockSpec((1, MW), lambda: (0, 0)),
            pl.BlockSpec((T, H2), lambda: (0, 0)),
        ),
    )(x, W_gate, expert_bias.reshape(1, E))


def _dispatch_body(xpk_hbm, p0_hbm, p1_hbm, xs_hbm, xrows, p0v, p1v,
                   sem0, sem1, sem2):
    wid = lax.axis_index("s") * NC + lax.axis_index("c")
    base = wid * TPW
    c0 = pltpu.async_copy(xpk_hbm.at[pl.ds(base, TPW)], xrows, sem0)
    c1 = pltpu.async_copy(p0_hbm.at[pl.ds(base, TPW)], p0v, sem1)
    c2 = pltpu.async_copy(p1_hbm.at[pl.ds(base, TPW)], p1v, sem2)
    c0.wait()
    c1.wait()
    c2.wait()
    a = pltpu.async_copy(xrows, xs_hbm.at[p0v], sem0)
    b = pltpu.async_copy(xrows, xs_hbm.at[p1v], sem1)
    a.wait()
    b.wait()


def _run_dispatch(xpk, pos0, pos1):
    mesh = plsc.VectorSubcoreMesh(core_axis_name="c", subcore_axis_name="s")
    f = functools.partial(
        pl.kernel,
        out_type=jax.ShapeDtypeStruct((NR, H2), jnp.int32),
        mesh=mesh,
        scratch_types=[
            pltpu.VMEM((TPW, H2), jnp.int32),
            pltpu.VMEM((TPW,), jnp.int32),
            pltpu.VMEM((TPW,), jnp.int32),
            pltpu.SemaphoreType.DMA,
            pltpu.SemaphoreType.DMA,
            pltpu.SemaphoreType.DMA,
        ],
    )(_dispatch_body)
    return f(xpk, pos0, pos1)


def _grouped_body(m_ref, xs_ref, wg_ref, wu_ref, wd_ref, ys_ref):
    b = pl.program_id(0)
    active = b < m_ref[0]

    @pl.when(active)
    def _():
        xa, xb_ = _unpack(xs_ref[...])
        hg = (jnp.dot(xa, wg_ref[0, :H2, :],
                      preferred_element_type=jnp.float32)
              + jnp.dot(xb_, wg_ref[0, H2:, :],
                        preferred_element_type=jnp.float32))
        hu = (jnp.dot(xa, wu_ref[0, :H2, :],
                      preferred_element_type=jnp.float32)
              + jnp.dot(xb_, wu_ref[0, H2:, :],
                        preferred_element_type=jnp.float32))
        mid = jax.nn.silu(hg) * hu
        ys = jnp.dot(mid, wd_ref[0], preferred_element_type=jnp.float32)
        ys_ref[...] = _pack(ys[:, :H2], ys[:, H2:])


def _run_grouped(meta1d, xs, Wg, Wu, Wd):
    grid_spec = pltpu.PrefetchScalarGridSpec(
        num_scalar_prefetch=1,
        grid=(NBLK,),
        in_specs=[
            pl.BlockSpec((BM, H2), lambda b, m: (b, 0)),
            pl.BlockSpec((1, H, INTER), lambda b, m: (m[b + 1], 0, 0)),
            pl.BlockSpec((1, H, INTER), lambda b, m: (m[b + 1], 0, 0)),
            pl.BlockSpec((1, INTER, H), lambda b, m: (m[b + 1], 0, 0)),
        ],
        out_specs=pl.BlockSpec((BM, H2), lambda b, m: (b, 0)),
    )
    return pl.pallas_call(
        _grouped_body,
        grid_spec=grid_spec,
        out_shape=jax.ShapeDtypeStruct((NR, H2), jnp.int32),
    )(meta1d, xs, Wg, Wu, Wd)


def _gather_body(ys_hbm, p0_hbm, p1_hbm, y0g_hbm, y1g_hbm,
                 y0b, y1b, p0v, p1v, sem0, sem1):
    wid = lax.axis_index("s") * NC + lax.axis_index("c")
    base = wid * TPW
    c0 = pltpu.async_copy(p0_hbm.at[pl.ds(base, TPW)], p0v, sem0)
    c1 = pltpu.async_copy(p1_hbm.at[pl.ds(base, TPW)], p1v, sem1)
    c0.wait()
    c1.wait()
    a = pltpu.async_copy(ys_hbm.at[p0v], y0b, sem0)
    b = pltpu.async_copy(ys_hbm.at[p1v], y1b, sem1)
    a.wait()
    c2 = pltpu.async_copy(y0b, y0g_hbm.at[pl.ds(base, TPW)], sem0)
    b.wait()
    c3 = pltpu.async_copy(y1b, y1g_hbm.at[pl.ds(base, TPW)], sem1)
    c2.wait()
    c3.wait()


def _run_gather(ys, pos0, pos1):
    mesh = plsc.VectorSubcoreMesh(core_axis_name="c", subcore_axis_name="s")
    f = functools.partial(
        pl.kernel,
        out_type=(
            jax.ShapeDtypeStruct((T, H2), jnp.int32),
            jax.ShapeDtypeStruct((T, H2), jnp.int32),
        ),
        mesh=mesh,
        scratch_types=[
            pltpu.VMEM((TPW, H2), jnp.int32),
            pltpu.VMEM((TPW, H2), jnp.int32),
            pltpu.VMEM((TPW,), jnp.int32),
            pltpu.VMEM((TPW,), jnp.int32),
            pltpu.SemaphoreType.DMA,
            pltpu.SemaphoreType.DMA,
        ],
    )(_gather_body)
    return f(ys, pos0, pos1)


def _shared_body(x_ref, wgs_ref, wus_ref, wds_ref, out_ref, wgsc, wusc, wdsc):
    @pl.when(pl.program_id(0) == 0)
    def _():
        wgsc[...] = wgs_ref[...].astype(jnp.bfloat16)
        wusc[...] = wus_ref[...].astype(jnp.bfloat16)
        wdsc[...] = wds_ref[...].astype(jnp.bfloat16)

    xb = x_ref[...].astype(jnp.bfloat16)
    hg = jnp.dot(xb, wgsc[...], preferred_element_type=jnp.float32)
    hu = jnp.dot(xb, wusc[...], preferred_element_type=jnp.float32)
    mid = (jax.nn.silu(hg) * hu).astype(jnp.bfloat16)
    out_ref[...] = jnp.dot(mid, wdsc[...], preferred_element_type=jnp.float32)


def _run_shared(x, Wg_s, Wu_s, Wd_s):
    return pl.pallas_call(
        _shared_body,
        grid=(T // TBLK,),
        out_shape=jax.ShapeDtypeStruct((T, H), jnp.float32),
        in_specs=[
            pl.BlockSpec((TBLK, H), lambda t: (t, 0)),
            pl.BlockSpec((H, SI), lambda t: (0, 0)),
            pl.BlockSpec((H, SI), lambda t: (0, 0)),
            pl.BlockSpec((SI, H), lambda t: (0, 0)),
        ],
        out_specs=pl.BlockSpec((TBLK, H), lambda t: (t, 0)),
        scratch_shapes=[
            pltpu.VMEM((H, SI), jnp.bfloat16),
            pltpu.VMEM((H, SI), jnp.bfloat16),
            pltpu.VMEM((SI, H), jnp.bfloat16),
        ],
    )(x, Wg_s, Wu_s, Wd_s)


def _combine_body(sh_ref, y0_ref, y1_ref, w0_ref, w1_ref, out_ref):
    sh = sh_ref[...]
    w0 = w0_ref[...][:, 0:1]
    w1 = w1_ref[...][:, 0:1]
    y0a, y0b = _unpack(y0_ref[...])
    y1a, y1b = _unpack(y1_ref[...])
    out_ref[:, :H2] = sh[:, :H2] + w0 * y0a + w1 * y1a
    out_ref[:, H2:] = sh[:, H2:] + w0 * y0b + w1 * y1b


def _run_combine(sh, y0g, y1g, w0, w1):
    return pl.pallas_call(
        _combine_body,
        grid=(T // TBLK,),
        out_shape=jax.ShapeDtypeStruct((T, H), jnp.float32),
        in_specs=[
            pl.BlockSpec((TBLK, H), lambda t: (t, 0)),
            pl.BlockSpec((TBLK, H2), lambda t: (t, 0)),
            pl.BlockSpec((TBLK, H2), lambda t: (t, 0)),
            pl.BlockSpec((TBLK, 16), lambda t: (t, 0)),
            pl.BlockSpec((TBLK, 16), lambda t: (t, 0)),
        ],
        out_specs=pl.BlockSpec((TBLK, H), lambda t: (t, 0)),
    )(sh, y0g, y1g, w0, w1)


def kernel(hidden_states, W_gate, Wg_s, Wu_s, Wd_s, Wg, Wu, Wd, expert_bias):
    b, s, h = hidden_states.shape
    x = hidden_states.reshape(T, H)

    pos0, pos1, w0, w1, meta, xpk = _run_router(x, W_gate, expert_bias)
    pos0 = pos0.reshape(T)
    pos1 = pos1.reshape(T)
    meta1d = meta.reshape(MW)

    sh = _run_shared(x, Wg_s, Wu_s, Wd_s)
    xs = _run_dispatch(xpk, pos0, pos1)
    ys = _run_grouped(meta1d, xs, Wg, Wu, Wd)
    y0g, y1g = _run_gather(ys, pos0, pos1)
    out = _run_combine(sh, y0g, y1g, w0, w1)
    return out.reshape(b, s, h)


# speedup vs baseline: 5.0893x; 1.0146x over previous
import functools

import jax
import jax.numpy as jnp
from jax import lax
from jax.experimental import pallas as pl
from jax.experimental.pallas import tpu as pltpu
from jax.experimental.pallas import tpu_sc as plsc

T = 2048
H = 1024
H2 = H // 2
E = 8
K = 2
INTER = 512
SI = 1024
BM = 512
NBLK = T * K // BM + E
NR = NBLK * BM
MW = NBLK + 1
NC = 2
NS = 16
NW = NC * NS
TPW = T // NW
TBLK = 512

def _pack(lo_f32, hi_f32):
    mask = jnp.full(lo_f32.shape, 0xFFFF0000, jnp.uint32)
    lo = pltpu.bitcast(lo_f32.astype(jnp.bfloat16).astype(jnp.float32),
                       jnp.uint32) >> 16
    hi = pltpu.bitcast(hi_f32.astype(jnp.bfloat16).astype(jnp.float32),
                       jnp.uint32) & mask
    return pltpu.bitcast(lo | hi, jnp.int32)


def _unpack(word_i32):
    mask = jnp.full(word_i32.shape, 0xFFFF0000, jnp.uint32)
    w = pltpu.bitcast(word_i32, jnp.uint32)
    lo = pltpu.bitcast(w << 16, jnp.float32)
    hi = pltpu.bitcast(w & mask, jnp.float32)
    return lo, hi


def _router_body(x_ref, wg_ref, b_ref, pos0_ref, pos1_ref, w0_ref, w1_ref,
                 meta_ref, xpk_ref):
    x = x_ref[...]
    xpk_ref[...] = _pack(x[:, :H2], x[:, H2:])
    scores = jax.nn.sigmoid(
        jnp.dot(x, wg_ref[...], preferred_element_type=jnp.float32))
    biased = scores + b_ref[...]
    iota = lax.broadcasted_iota(jnp.int32, (T, E), 1)
    m0 = jnp.max(biased, axis=1, keepdims=True)
    sel0 = jnp.min(jnp.where(biased >= m0, iota, E), axis=1, keepdims=True)
    neg = jnp.where(iota == sel0, -jnp.inf, biased)
    m1 = jnp.max(neg, axis=1, keepdims=True)
    sel1 = jnp.min(jnp.where(neg >= m1, iota, E), axis=1, keepdims=True)
    s0 = jnp.sum(jnp.where(iota == sel0, scores, 0.0), axis=1, keepdims=True)
    s1 = jnp.sum(jnp.where(iota == sel1, scores, 0.0), axis=1, keepdims=True)
    denom = s0 + s1 + 1e-20
    w0_ref[...] = jnp.broadcast_to(s0 / denom, (T, 16))
    w1_ref[...] = jnp.broadcast_to(s1 / denom, (T, 16))

    csum = jnp.logical_or(iota == sel0, iota == sel1).astype(jnp.float32)
    shift = 1
    while shift < T:
        shifted = jnp.concatenate(
            [jnp.zeros((shift, E), jnp.float32), csum[:T - shift]], axis=0)
        csum = csum + shifted
        shift *= 2
    counts = csum[T - 1:T, :]
    cnt_pad = jnp.floor((counts + (BM - 1)) / BM) * BM
    er = lax.broadcasted_iota(jnp.int32, (E, E), 0)
    ec = lax.broadcasted_iota(jnp.int32, (E, E), 1)
    upper = (er < ec).astype(jnp.float32)
    offs = jnp.dot(cnt_pad, upper, preferred_element_type=jnp.float32)

    posf0 = jnp.sum(jnp.where(iota == sel0, offs + csum - 1.0, 0.0),
                    axis=1, keepdims=True)
    posf1 = jnp.sum(jnp.where(iota == sel1, offs + csum - 1.0, 0.0),
                    axis=1, keepdims=True)
    pos0_ref[...] = posf0.astype(jnp.int32)
    pos1_ref[...] = posf1.astype(jnp.int32)

    evec = lax.broadcasted_iota(jnp.int32, (1, E), 1)
    off_s = [jnp.sum(jnp.where(evec == e, offs, 0.0)) for e in range(E)]
    cnt_s = [jnp.sum(jnp.where(evec == e, cnt_pad, 0.0)) for e in range(E)]
    nb_total = ((off_s[E - 1] + cnt_s[E - 1]) / BM).astype(jnp.int32)
    bio = lax.broadcasted_iota(jnp.int32, (1, MW), 1)
    bvals = (bio - 1) * BM
    be = sum(((bvals.astype(jnp.float32) >= off_s[e]).astype(jnp.int32))
             for e in range(E)) - 1
    meta_ref[...] = jnp.where(bio == 0, nb_total, be)


def _run_router(x, W_gate, expert_bias):
    return pl.pallas_call(
        _router_body,
        out_shape=(
            jax.ShapeDtypeStruct((T, 1), jnp.int32),
            jax.ShapeDtypeStruct((T, 1), jnp.int32),
            jax.ShapeDtypeStruct((T, 16), jnp.float32),
            jax.ShapeDtypeStruct((T, 16), jnp.float32),
            jax.ShapeDtypeStruct((1, MW), jnp.int32),
            jax.ShapeDtypeStruct((T, H2), jnp.int32),
        ),
        in_specs=[
            pl.BlockSpec((T, H), lambda: (0, 0)),
            pl.BlockSpec((H, E), lambda: (0, 0)),
            pl.BlockSpec((1, E), lambda: (0, 0)),
        ],
        out_specs=(
            pl.BlockSpec((T, 1), lambda: (0, 0)),
            pl.BlockSpec((T, 1), lambda: (0, 0)),
            pl.BlockSpec((T, 16), lambda: (0, 0)),
            pl.BlockSpec((T, 16), lambda: (0, 0)),
            pl.BlockSpec((1, MW), lambda: (0, 0)),
            pl.BlockSpec((T, H2), lambda: (0, 0)),
        ),
    )(x, W_gate, expert_bias.reshape(1, E))


def _dispatch_body(xpk_hbm, p0_hbm, p1_hbm, xs_hbm, xrows, p0v, p1v,
                   sem0, sem1, sem2):
    wid = lax.axis_index("s") * NC + lax.axis_index("c")
    base = wid * TPW
    c0 = pltpu.async_copy(xpk_hbm.at[pl.ds(base, TPW)], xrows, sem0)
    c1 = pltpu.async_copy(p0_hbm.at[pl.ds(base, TPW)], p0v, sem1)
    c2 = pltpu.async_copy(p1_hbm.at[pl.ds(base, TPW)], p1v, sem2)
    c0.wait()
    c1.wait()
    c2.wait()
    a = pltpu.async_copy(xrows, xs_hbm.at[p0v], sem0)
    b = pltpu.async_copy(xrows, xs_hbm.at[p1v], sem1)
    a.wait()
    b.wait()


def _run_dispatch(xpk, pos0, pos1):
    mesh = plsc.VectorSubcoreMesh(core_axis_name="c", subcore_axis_name="s")
    f = functools.partial(
        pl.kernel,
        out_type=jax.ShapeDtypeStruct((NR, H2), jnp.int32),
        mesh=mesh,
        scratch_types=[
            pltpu.VMEM((TPW, H2), jnp.int32),
            pltpu.VMEM((TPW,), jnp.int32),
            pltpu.VMEM((TPW,), jnp.int32),
            pltpu.SemaphoreType.DMA,
            pltpu.SemaphoreType.DMA,
            pltpu.SemaphoreType.DMA,
        ],
    )(_dispatch_body)
    return f(xpk, pos0, pos1)


def _grouped_body(m_ref, xs_ref, wg_ref, wu_ref, wd_ref, ys_ref):
    b = pl.program_id(0)
    active = b < m_ref[0]

    @pl.when(active)
    def _():
        xa, xb_ = _unpack(xs_ref[...])
        hg = (jnp.dot(xa, wg_ref[0, :H2, :],
                      preferred_element_type=jnp.float32)
              + jnp.dot(xb_, wg_ref[0, H2:, :],
                        preferred_element_type=jnp.float32))
        hu = (jnp.dot(xa, wu_ref[0, :H2, :],
                      preferred_element_type=jnp.float32)
              + jnp.dot(xb_, wu_ref[0, H2:, :],
                        preferred_element_type=jnp.float32))
        mid = jax.nn.silu(hg) * hu
        ys = jnp.dot(mid, wd_ref[0], preferred_element_type=jnp.float32)
        ys_ref[...] = _pack(ys[:, :H2], ys[:, H2:])


def _run_grouped(meta1d, xs, Wg, Wu, Wd):
    grid_spec = pltpu.PrefetchScalarGridSpec(
        num_scalar_prefetch=1,
        grid=(NBLK,),
        in_specs=[
            pl.BlockSpec((BM, H2), lambda b, m: (b, 0)),
            pl.BlockSpec((1, H, INTER), lambda b, m: (m[b + 1], 0, 0)),
            pl.BlockSpec((1, H, INTER), lambda b, m: (m[b + 1], 0, 0)),
            pl.BlockSpec((1, INTER, H), lambda b, m: (m[b + 1], 0, 0)),
        ],
        out_specs=pl.BlockSpec((BM, H2), lambda b, m: (b, 0)),
    )
    return pl.pallas_call(
        _grouped_body,
        grid_spec=grid_spec,
        out_shape=jax.ShapeDtypeStruct((NR, H2), jnp.int32),
    )(meta1d, xs, Wg, Wu, Wd)


def _gather_body(ys_hbm, p0_hbm, p1_hbm, y0g_hbm, y1g_hbm,
                 y0b, y1b, p0v, p1v, sem0, sem1):
    wid = lax.axis_index("s") * NC + lax.axis_index("c")
    base = wid * TPW
    c0 = pltpu.async_copy(p0_hbm.at[pl.ds(base, TPW)], p0v, sem0)
    c1 = pltpu.async_copy(p1_hbm.at[pl.ds(base, TPW)], p1v, sem1)
    c0.wait()
    c1.wait()
    a = pltpu.async_copy(ys_hbm.at[p0v], y0b, sem0)
    b = pltpu.async_copy(ys_hbm.at[p1v], y1b, sem1)
    a.wait()
    c2 = pltpu.async_copy(y0b, y0g_hbm.at[pl.ds(base, TPW)], sem0)
    b.wait()
    c3 = pltpu.async_copy(y1b, y1g_hbm.at[pl.ds(base, TPW)], sem1)
    c2.wait()
    c3.wait()


def _run_gather(ys, pos0, pos1):
    mesh = plsc.VectorSubcoreMesh(core_axis_name="c", subcore_axis_name="s")
    f = functools.partial(
        pl.kernel,
        out_type=(
            jax.ShapeDtypeStruct((T, H2), jnp.int32),
            jax.ShapeDtypeStruct((T, H2), jnp.int32),
        ),
        mesh=mesh,
        scratch_types=[
            pltpu.VMEM((TPW, H2), jnp.int32),
            pltpu.VMEM((TPW, H2), jnp.int32),
            pltpu.VMEM((TPW,), jnp.int32),
            pltpu.VMEM((TPW,), jnp.int32),
            pltpu.SemaphoreType.DMA,
            pltpu.SemaphoreType.DMA,
        ],
    )(_gather_body)
    return f(ys, pos0, pos1)


def _shared_body(xpk_ref, wgs_ref, wus_ref, wds_ref, out_ref):
    xa, xb_ = _unpack(xpk_ref[...])
    hg = (jnp.dot(xa, wgs_ref[:H2, :], preferred_element_type=jnp.float32)
          + jnp.dot(xb_, wgs_ref[H2:, :], preferred_element_type=jnp.float32))
    hu = (jnp.dot(xa, wus_ref[:H2, :], preferred_element_type=jnp.float32)
          + jnp.dot(xb_, wus_ref[H2:, :], preferred_element_type=jnp.float32))
    mid = jax.nn.silu(hg) * hu
    out_ref[...] = jnp.dot(mid, wds_ref[...], preferred_element_type=jnp.float32)


def _run_shared(xpk, Wg_s, Wu_s, Wd_s):
    return pl.pallas_call(
        _shared_body,
        grid=(T // TBLK,),
        out_shape=jax.ShapeDtypeStruct((T, H), jnp.float32),
        in_specs=[
            pl.BlockSpec((TBLK, H2), lambda t: (t, 0)),
            pl.BlockSpec((H, SI), lambda t: (0, 0)),
            pl.BlockSpec((H, SI), lambda t: (0, 0)),
            pl.BlockSpec((SI, H), lambda t: (0, 0)),
        ],
        out_specs=pl.BlockSpec((TBLK, H), lambda t: (t, 0)),
    )(xpk, Wg_s, Wu_s, Wd_s)


def _combine_body(sh_ref, y0_ref, y1_ref, w0_ref, w1_ref, out_ref):
    sh = sh_ref[...]
    w0 = w0_ref[...][:, 0:1]
    w1 = w1_ref[...][:, 0:1]
    y0a, y0b = _unpack(y0_ref[...])
    y1a, y1b = _unpack(y1_ref[...])
    out_ref[:, :H2] = sh[:, :H2] + w0 * y0a + w1 * y1a
    out_ref[:, H2:] = sh[:, H2:] + w0 * y0b + w1 * y1b


def _run_combine(sh, y0g, y1g, w0, w1):
    return pl.pallas_call(
        _combine_body,
        grid=(T // TBLK,),
        out_shape=jax.ShapeDtypeStruct((T, H), jnp.float32),
        in_specs=[
            pl.BlockSpec((TBLK, H), lambda t: (t, 0)),
            pl.BlockSpec((TBLK, H2), lambda t: (t, 0)),
            pl.BlockSpec((TBLK, H2), lambda t: (t, 0)),
            pl.BlockSpec((TBLK, 16), lambda t: (t, 0)),
            pl.BlockSpec((TBLK, 16), lambda t: (t, 0)),
        ],
        out_specs=pl.BlockSpec((TBLK, H), lambda t: (t, 0)),
    )(sh, y0g, y1g, w0, w1)


def kernel(hidden_states, W_gate, Wg_s, Wu_s, Wd_s, Wg, Wu, Wd, expert_bias):
    b, s, h = hidden_states.shape
    x = hidden_states.reshape(T, H)

    pos0, pos1, w0, w1, meta, xpk = _run_router(x, W_gate, expert_bias)
    pos0 = pos0.reshape(T)
    pos1 = pos1.reshape(T)
    meta1d = meta.reshape(MW)

    sh = _run_shared(xpk, Wg_s, Wu_s, Wd_s)
    xs = _run_dispatch(xpk, pos0, pos1)
    ys = _run_grouped(meta1d, xs, Wg, Wu, Wd)
    y0g, y1g = _run_gather(ys, pos0, pos1)
    out = _run_combine(sh, y0g, y1g, w0, w1)
    return out.reshape(b, s, h)


# speedup vs baseline: 5.4389x; 1.0687x over previous
import functools

import jax
import jax.numpy as jnp
from jax import lax
from jax.experimental import pallas as pl
from jax.experimental.pallas import tpu as pltpu
from jax.experimental.pallas import tpu_sc as plsc

T = 2048
H = 1024
H2 = H // 2
E = 8
K = 2
INTER = 512
SI = 1024
BM = 512
NBLK = T * K // BM + E
NR = NBLK * BM
MW = NBLK + 1
NC = 2
NS = 16
NW = NC * NS
TPW = T // NW
TBLK = 512

def _pack(lo_f32, hi_f32):
    mask = jnp.full(lo_f32.shape, 0xFFFF0000, jnp.uint32)
    lo = pltpu.bitcast(lo_f32.astype(jnp.bfloat16).astype(jnp.float32),
                       jnp.uint32) >> 16
    hi = pltpu.bitcast(hi_f32.astype(jnp.bfloat16).astype(jnp.float32),
                       jnp.uint32) & mask
    return pltpu.bitcast(lo | hi, jnp.int32)


def _unpack(word_i32):
    mask = jnp.full(word_i32.shape, 0xFFFF0000, jnp.uint32)
    w = pltpu.bitcast(word_i32, jnp.uint32)
    lo = pltpu.bitcast(w << 16, jnp.float32)
    hi = pltpu.bitcast(w & mask, jnp.float32)
    return lo, hi


def _router_body(x_ref, wg_ref, b_ref, pos0_ref, pos1_ref, w0_ref, w1_ref,
                 meta_ref, xpk_ref):
    x = x_ref[...]
    xpk_ref[...] = _pack(x[:, :H2], x[:, H2:])
    scores = jax.nn.sigmoid(
        jnp.dot(x, wg_ref[...], preferred_element_type=jnp.float32))
    biased = scores + b_ref[...]
    iota = lax.broadcasted_iota(jnp.int32, (T, E), 1)
    m0 = jnp.max(biased, axis=1, keepdims=True)
    sel0 = jnp.min(jnp.where(biased >= m0, iota, E), axis=1, keepdims=True)
    neg = jnp.where(iota == sel0, -jnp.inf, biased)
    m1 = jnp.max(neg, axis=1, keepdims=True)
    sel1 = jnp.min(jnp.where(neg >= m1, iota, E), axis=1, keepdims=True)
    s0 = jnp.sum(jnp.where(iota == sel0, scores, 0.0), axis=1, keepdims=True)
    s1 = jnp.sum(jnp.where(iota == sel1, scores, 0.0), axis=1, keepdims=True)
    denom = s0 + s1 + 1e-20
    w0_ref[...] = jnp.broadcast_to(s0 / denom, (T, 16))
    w1_ref[...] = jnp.broadcast_to(s1 / denom, (T, 16))

    csum = jnp.logical_or(iota == sel0, iota == sel1).astype(jnp.float32)
    shift = 1
    while shift < T:
        shifted = jnp.concatenate(
            [jnp.zeros((shift, E), jnp.float32), csum[:T - shift]], axis=0)
        csum = csum + shifted
        shift *= 2
    counts = csum[T - 1:T, :]
    cnt_pad = jnp.floor((counts + (BM - 1)) / BM) * BM
    er = lax.broadcasted_iota(jnp.int32, (E, E), 0)
    ec = lax.broadcasted_iota(jnp.int32, (E, E), 1)
    upper = (er < ec).astype(jnp.float32)
    offs = jnp.dot(cnt_pad, upper, preferred_element_type=jnp.float32)

    posf0 = jnp.sum(jnp.where(iota == sel0, offs + csum - 1.0, 0.0),
                    axis=1, keepdims=True)
    posf1 = jnp.sum(jnp.where(iota == sel1, offs + csum - 1.0, 0.0),
                    axis=1, keepdims=True)
    pos0_ref[...] = posf0.astype(jnp.int32).reshape(T)
    pos1_ref[...] = posf1.astype(jnp.int32).reshape(T)

    evec = lax.broadcasted_iota(jnp.int32, (1, E), 1)
    off_s = [jnp.sum(jnp.where(evec == e, offs, 0.0)) for e in range(E)]
    cnt_s = [jnp.sum(jnp.where(evec == e, cnt_pad, 0.0)) for e in range(E)]
    nb_total = ((off_s[E - 1] + cnt_s[E - 1]) / BM).astype(jnp.int32)
    bio = lax.broadcasted_iota(jnp.int32, (1, MW), 1)
    bvals = (bio - 1) * BM
    be = sum(((bvals.astype(jnp.float32) >= off_s[e]).astype(jnp.int32))
             for e in range(E)) - 1
    meta_ref[...] = jnp.where(bio == 0, nb_total, be)


def _run_router(x, W_gate, expert_bias):
    return pl.pallas_call(
        _router_body,
        out_shape=(
            jax.ShapeDtypeStruct((T,), jnp.int32),
            jax.ShapeDtypeStruct((T,), jnp.int32),
            jax.ShapeDtypeStruct((T, 16), jnp.float32),
            jax.ShapeDtypeStruct((T, 16), jnp.float32),
            jax.ShapeDtypeStruct((1, MW), jnp.int32),
            jax.ShapeDtypeStruct((T, H2), jnp.int32),
        ),
        in_specs=[
            pl.BlockSpec((T, H), lambda: (0, 0)),
            pl.BlockSpec((H, E), lambda: (0, 0)),
            pl.BlockSpec((1, E), lambda: (0, 0)),
        ],
        out_specs=(
            pl.BlockSpec((T,), lambda: (0,)),
            pl.BlockSpec((T,), lambda: (0,)),
            pl.BlockSpec((T, 16), lambda: (0, 0)),
            pl.BlockSpec((T, 16), lambda: (0, 0)),
            pl.BlockSpec((1, MW), lambda: (0, 0)),
            pl.BlockSpec((T, H2), lambda: (0, 0)),
        ),
    )(x, W_gate, expert_bias.reshape(1, E))


def _dispatch_body(xpk_hbm, p0_hbm, p1_hbm, xs_hbm, xrows, p0v, p1v,
                   sem0, sem1, sem2):
    wid = lax.axis_index("s") * NC + lax.axis_index("c")
    base = wid * TPW
    c0 = pltpu.async_copy(xpk_hbm.at[pl.ds(base, TPW)], xrows, sem0)
    c1 = pltpu.async_copy(p0_hbm.at[pl.ds(base, TPW)], p0v, sem1)
    c2 = pltpu.async_copy(p1_hbm.at[pl.ds(base, TPW)], p1v, sem2)
    c0.wait()
    c1.wait()
    c2.wait()
    a = pltpu.async_copy(xrows, xs_hbm.at[p0v], sem0)
    b = pltpu.async_copy(xrows, xs_hbm.at[p1v], sem1)
    a.wait()
    b.wait()


def _run_dispatch(xpk, pos0, pos1):
    mesh = plsc.VectorSubcoreMesh(core_axis_name="c", subcore_axis_name="s")
    f = functools.partial(
        pl.kernel,
        out_type=jax.ShapeDtypeStruct((NR, H2), jnp.int32),
        mesh=mesh,
        scratch_types=[
            pltpu.VMEM((TPW, H2), jnp.int32),
            pltpu.VMEM((TPW,), jnp.int32),
            pltpu.VMEM((TPW,), jnp.int32),
            pltpu.SemaphoreType.DMA,
            pltpu.SemaphoreType.DMA,
            pltpu.SemaphoreType.DMA,
        ],
    )(_dispatch_body)
    return f(xpk, pos0, pos1)


def _grouped_body(m_ref, xs_ref, wg_ref, wu_ref, wd_ref, ys_ref):
    b = pl.program_id(0)
    active = b < m_ref[0]

    @pl.when(active)
    def _():
        xa, xb_ = _unpack(xs_ref[...])
        hg = (jnp.dot(xa, wg_ref[0, :H2, :],
                      preferred_element_type=jnp.float32)
              + jnp.dot(xb_, wg_ref[0, H2:, :],
                        preferred_element_type=jnp.float32))
        hu = (jnp.dot(xa, wu_ref[0, :H2, :],
                      preferred_element_type=jnp.float32)
              + jnp.dot(xb_, wu_ref[0, H2:, :],
                        preferred_element_type=jnp.float32))
        mid = jax.nn.silu(hg) * hu
        ys = jnp.dot(mid, wd_ref[0], preferred_element_type=jnp.float32)
        ys_ref[...] = _pack(ys[:, :H2], ys[:, H2:])


def _run_grouped(meta1d, xs, Wg, Wu, Wd):
    grid_spec = pltpu.PrefetchScalarGridSpec(
        num_scalar_prefetch=1,
        grid=(NBLK,),
        in_specs=[
            pl.BlockSpec((BM, H2),
                         lambda b, m: (jnp.where(b < m[0], b, m[0] - 1), 0)),
            pl.BlockSpec((1, H, INTER), lambda b, m: (m[b + 1], 0, 0)),
            pl.BlockSpec((1, H, INTER), lambda b, m: (m[b + 1], 0, 0)),
            pl.BlockSpec((1, INTER, H), lambda b, m: (m[b + 1], 0, 0)),
        ],
        out_specs=pl.BlockSpec(
            (BM, H2), lambda b, m: (jnp.where(b < m[0], b, m[0] - 1), 0)),
    )
    return pl.pallas_call(
        _grouped_body,
        grid_spec=grid_spec,
        out_shape=jax.ShapeDtypeStruct((NR, H2), jnp.int32),
    )(meta1d, xs, Wg, Wu, Wd)


def _gather_body(ys_hbm, p0_hbm, p1_hbm, y0g_hbm, y1g_hbm,
                 y0b, y1b, p0v, p1v, sem0, sem1):
    wid = lax.axis_index("s") * NC + lax.axis_index("c")
    base = wid * TPW
    c0 = pltpu.async_copy(p0_hbm.at[pl.ds(base, TPW)], p0v, sem0)
    c1 = pltpu.async_copy(p1_hbm.at[pl.ds(base, TPW)], p1v, sem1)
    c0.wait()
    c1.wait()
    a = pltpu.async_copy(ys_hbm.at[p0v], y0b, sem0)
    b = pltpu.async_copy(ys_hbm.at[p1v], y1b, sem1)
    a.wait()
    c2 = pltpu.async_copy(y0b, y0g_hbm.at[pl.ds(base, TPW)], sem0)
    b.wait()
    c3 = pltpu.async_copy(y1b, y1g_hbm.at[pl.ds(base, TPW)], sem1)
    c2.wait()
    c3.wait()


def _run_gather(ys, pos0, pos1):
    mesh = plsc.VectorSubcoreMesh(core_axis_name="c", subcore_axis_name="s")
    f = functools.partial(
        pl.kernel,
        out_type=(
            jax.ShapeDtypeStruct((T, H2), jnp.int32),
            jax.ShapeDtypeStruct((T, H2), jnp.int32),
        ),
        mesh=mesh,
        scratch_types=[
            pltpu.VMEM((TPW, H2), jnp.int32),
            pltpu.VMEM((TPW, H2), jnp.int32),
            pltpu.VMEM((TPW,), jnp.int32),
            pltpu.VMEM((TPW,), jnp.int32),
            pltpu.SemaphoreType.DMA,
            pltpu.SemaphoreType.DMA,
        ],
    )(_gather_body)
    return f(ys, pos0, pos1)


def _shared_body(xpk_ref, wgs_ref, wus_ref, wds_ref, out_ref):
    xa, xb_ = _unpack(xpk_ref[...])
    hg = (jnp.dot(xa, wgs_ref[:H2, :], preferred_element_type=jnp.float32)
          + jnp.dot(xb_, wgs_ref[H2:, :], preferred_element_type=jnp.float32))
    hu = (jnp.dot(xa, wus_ref[:H2, :], preferred_element_type=jnp.float32)
          + jnp.dot(xb_, wus_ref[H2:, :], preferred_element_type=jnp.float32))
    mid = jax.nn.silu(hg) * hu
    out_ref[...] = jnp.dot(mid, wds_ref[...], preferred_element_type=jnp.float32)


def _run_shared(xpk, Wg_s, Wu_s, Wd_s):
    return pl.pallas_call(
        _shared_body,
        grid=(T // TBLK,),
        out_shape=jax.ShapeDtypeStruct((T, H), jnp.float32),
        in_specs=[
            pl.BlockSpec((TBLK, H2), lambda t: (t, 0)),
            pl.BlockSpec((H, SI), lambda t: (0, 0)),
            pl.BlockSpec((H, SI), lambda t: (0, 0)),
            pl.BlockSpec((SI, H), lambda t: (0, 0)),
        ],
        out_specs=pl.BlockSpec((TBLK, H), lambda t: (t, 0)),
    )(xpk, Wg_s, Wu_s, Wd_s)


def _combine_body(sh_ref, y0_ref, y1_ref, w0_ref, w1_ref, out_ref):
    sh = sh_ref[...]
    w0 = w0_ref[...][:, 0:1]
    w1 = w1_ref[...][:, 0:1]
    y0a, y0b = _unpack(y0_ref[...])
    y1a, y1b = _unpack(y1_ref[...])
    out_ref[:, :H2] = sh[:, :H2] + w0 * y0a + w1 * y1a
    out_ref[:, H2:] = sh[:, H2:] + w0 * y0b + w1 * y1b


def _run_combine(sh, y0g, y1g, w0, w1):
    return pl.pallas_call(
        _combine_body,
        grid=(T // TBLK,),
        out_shape=jax.ShapeDtypeStruct((T, H), jnp.float32),
        in_specs=[
            pl.BlockSpec((TBLK, H), lambda t: (t, 0)),
            pl.BlockSpec((TBLK, H2), lambda t: (t, 0)),
            pl.BlockSpec((TBLK, H2), lambda t: (t, 0)),
            pl.BlockSpec((TBLK, 16), lambda t: (t, 0)),
            pl.BlockSpec((TBLK, 16), lambda t: (t, 0)),
        ],
        out_specs=pl.BlockSpec((TBLK, H), lambda t: (t, 0)),
    )(sh, y0g, y1g, w0, w1)


def kernel(hidden_states, W_gate, Wg_s, Wu_s, Wd_s, Wg, Wu, Wd, expert_bias):
    b, s, h = hidden_states.shape
    x = hidden_states.reshape(T, H)

    pos0, pos1, w0, w1, meta, xpk = _run_router(x, W_gate, expert_bias)
    meta1d = meta.reshape(MW)

    sh = _run_shared(xpk, Wg_s, Wu_s, Wd_s)
    xs = _run_dispatch(xpk, pos0, pos1)
    ys = _run_grouped(meta1d, xs, Wg, Wu, Wd)
    y0g, y1g = _run_gather(ys, pos0, pos1)
    out = _run_combine(sh, y0g, y1g, w0, w1)
    return out.reshape(b, s, h)
